# Initial kernel scaffold; baseline (speedup 1.0000x reference)
import jax
import jax.numpy as jnp
from jax.experimental import pallas as pl


def kernel(Gu, Gi, edge_features, Wu, bu, Wi, bi, L0, rows, cols):
    raise NotImplementedError("write your pallas kernel here")


# speedup vs baseline: 3.4450x; 3.4450x over previous
import functools

import jax
import jax.numpy as jnp
from jax import lax
from jax.experimental import pallas as pl
from jax.experimental.pallas import tpu as pltpu
from jax.experimental.pallas import tpu_sc as plsc

NU = 25000
PAD_HALF = 25600
XP = 2 * PAD_HALF
KD = 64
EH = 400000
EP = 401408
CH = 128
K1_CH = EP // (32 * CH)
K3_CH = EP // (16 * CH)
ACC_STRIPE = PAD_HALF // 16

_f32 = jnp.float32
_i32 = jnp.int32


def _p2_tc(ef_p, Wu, bu, Wi, bi):
    nblk = EP // 2048

    def body(ef_ref, wu_ref, bu_ref, wi_ref, bi_ref, pu_ref, pi_ref):
        e = ef_ref[...]
        pu = jnp.dot(e, wu_ref[...], preferred_element_type=_f32) + bu_ref[...]
        pi = jnp.dot(e, wi_ref[...], preferred_element_type=_f32) + bi_ref[...]
        pu_ref[...] = pu * pu
        pi_ref[...] = pi * pi

    return pl.pallas_call(
        body,
        grid=(nblk,),
        in_specs=[
            pl.BlockSpec((2048, 16), lambda i: (i, 0)),
            pl.BlockSpec((16, KD), lambda i: (0, 0)),
            pl.BlockSpec((1, KD), lambda i: (0, 0)),
            pl.BlockSpec((16, KD), lambda i: (0, 0)),
            pl.BlockSpec((1, KD), lambda i: (0, 0)),
        ],
        out_specs=[pl.BlockSpec((2048, KD), lambda i: (i, 0))] * 2,
        out_shape=[jax.ShapeDtypeStruct((EP, KD), _f32)] * 2,
    )(ef_p, Wu, bu.reshape(1, KD), Wi, bi.reshape(1, KD))


def _dis_tc(degp):
    def body(d_ref, o_ref):
        d = jnp.sum(d_ref[...], axis=0)
        o_ref[...] = jnp.where(d > 0, 1.0 / jnp.sqrt(d), 0.0)

    out = pl.pallas_call(
        body,
        out_shape=jax.ShapeDtypeStruct((XP // 128, 128), _f32),
    )(degp.reshape(32, XP // 128, 128))
    return out.reshape(XP, 1)


def _rowscale_tc(x, dis2d):
    nblk = XP // 2048

    def body(x_ref, d_ref, o_ref):
        o_ref[...] = x_ref[...] * d_ref[...]

    return pl.pallas_call(
        body,
        grid=(nblk,),
        in_specs=[
            pl.BlockSpec((2048, KD), lambda i: (i, 0)),
            pl.BlockSpec((2048, 1), lambda i: (i, 0)),
        ],
        out_specs=pl.BlockSpec((2048, KD), lambda i: (i, 0)),
        out_shape=jax.ShapeDtypeStruct((XP, KD), _f32),
    )(x, dis2d)


def _k1_sc(x, p2u, p2i, r1, bix, cu, l0u, l0i):
    mesh = plsc.VectorSubcoreMesh(core_axis_name="c", subcore_axis_name="s")

    @functools.partial(
        pl.kernel,
        out_type=[
            jax.ShapeDtypeStruct((2, EP), _f32),
            jax.ShapeDtypeStruct((32, XP), _f32),
        ],
        mesh=mesh,
        compiler_params=pltpu.CompilerParams(
            needs_layout_passes=False, use_tc_tiling_on_sc=False),
        scratch_types=[
            pltpu.VMEM((CH, KD), _f32),
            pltpu.VMEM((CH, KD), _f32),
            pltpu.VMEM((CH, KD), _f32),
            pltpu.VMEM((CH, KD), _f32),
            pltpu.VMEM((CH,), _i32),
            pltpu.VMEM((CH,), _i32),
            pltpu.VMEM((CH,), _i32),
            pltpu.VMEM((CH,), _f32),
            pltpu.VMEM((CH,), _f32),
            pltpu.VMEM((CH,), _f32),
            pltpu.VMEM((CH,), _f32),
            pltpu.VMEM((XP,), _f32),
            pltpu.SemaphoreType.DMA,
        ],
    )
    def k1(x_h, pu_h, pi_h, r1_h, bix_h, cu_h, l0u_h, l0i_h,
           keep_h, degp_h,
           a_v, b_v, pu_v, pi_v, ri_v, bi_v, ci_v, lu_v, li_v,
           ku_v, ki_v, deg_v, sem):
        c = lax.axis_index("c")
        s = lax.axis_index("s")
        wid = c * 16 + s
        iota = lax.iota(_i32, 16)
        zero16 = jnp.zeros((16,), _f32)

        def zdeg(i, _):
            deg_v[pl.ds(i * 16, 16)] = zero16
            return 0

        lax.fori_loop(0, XP // 16, zdeg, 0)

        base = wid * (K1_CH * CH)

        def chunk(co, _):
            eoff = base + co * CH
            pltpu.sync_copy(r1_h.at[pl.ds(eoff, CH)], ri_v)
            pltpu.sync_copy(bix_h.at[pl.ds(eoff, CH)], bi_v)
            pltpu.sync_copy(cu_h.at[pl.ds(eoff, CH)], ci_v)
            pltpu.sync_copy(l0u_h.at[pl.ds(eoff, CH)], lu_v)
            pltpu.sync_copy(l0i_h.at[pl.ds(eoff, CH)], li_v)
            cp_a = pltpu.async_copy(x_h.at[ri_v], a_v, sem)
            cp_b = pltpu.async_copy(x_h.at[bi_v], b_v, sem)
            cp_u = pltpu.async_copy(pu_h.at[pl.ds(eoff, CH)], pu_v, sem)
            cp_i = pltpu.async_copy(pi_h.at[pl.ds(eoff, CH)], pi_v, sem)
            cp_a.wait()
            cp_b.wait()
            cp_u.wait()
            cp_i.wait()
            for g in range(CH // 16):
                rowi = g * 16 + iota

                def dot_k(k, acc):
                    abu, aau, bbu, abi, aai, bbi = acc
                    ck = jnp.full((16,), 0, _i32) + k
                    va = plsc.load_gather(a_v, [rowi, ck])
                    vb = plsc.load_gather(b_v, [rowi, ck])
                    vu = plsc.load_gather(pu_v, [rowi, ck])
                    vi = plsc.load_gather(pi_v, [rowi, ck])
                    ab = va * vb
                    aa = va * va
                    bb = vb * vb
                    return (abu + ab * vu, aau + aa * vu, bbu + bb * vu,
                            abi + ab * vi, aai + aa * vi, bbi + bb * vi)

                z = jnp.zeros((16,), _f32)
                abu, aau, bbu, abi, aai, bbi = lax.fori_loop(
                    0, KD, dot_k, (z, z, z, z, z, z))
                tl_u = 0.2 - lu_v[pl.ds(g * 16, 16)]
                tl_i = 0.2 - li_v[pl.ds(g * 16, 16)]
                e2 = jnp.float32(1e-16)
                ku = jnp.where(
                    (tl_u <= 0)
                    | ((abu > 0)
                       & (abu * abu >= tl_u * tl_u * jnp.maximum(aau, e2)
                          * jnp.maximum(bbu, e2))),
                    1.0, 0.0)
                ki = jnp.where(
                    (tl_i <= 0)
                    | ((abi > 0)
                       & (abi * abi >= tl_i * tl_i * jnp.maximum(aai, e2)
                          * jnp.maximum(bbi, e2))),
                    1.0, 0.0)
                valid = (eoff + g * 16 + iota) < EH
                ku = jnp.where(valid, ku, 0.0)
                ki = jnp.where(valid, ki, 0.0)
                ku_v[pl.ds(g * 16, 16)] = ku
                ki_v[pl.ds(g * 16, 16)] = ki
                plsc.addupdate_scatter(deg_v, [ri_v[pl.ds(g * 16, 16)]], ku)
                plsc.addupdate_scatter(deg_v, [ci_v[pl.ds(g * 16, 16)]], ki)
            pltpu.sync_copy(ku_v, keep_h.at[0, pl.ds(eoff, CH)])
            pltpu.sync_copy(ki_v, keep_h.at[1, pl.ds(eoff, CH)])
            return 0

        lax.fori_loop(0, K1_CH, chunk, 0)
        pltpu.sync_copy(deg_v, degp_h.at[wid])

    return k1(x, p2u, p2i, r1, bix, cu, l0u, l0i)


def _k3_sc(xs, keep, rowloc, colglb):
    mesh = plsc.VectorSubcoreMesh(core_axis_name="c", subcore_axis_name="s")

    @functools.partial(
        pl.kernel,
        out_type=jax.ShapeDtypeStruct((XP, KD), _f32),
        mesh=mesh,
        compiler_params=pltpu.CompilerParams(
            needs_layout_passes=False, use_tc_tiling_on_sc=False),
        scratch_types=[
            pltpu.VMEM((CH, KD), _f32),
            pltpu.VMEM((CH, KD), _f32),
            pltpu.VMEM((CH,), _i32),
            pltpu.VMEM((CH,), _i32),
            pltpu.VMEM((CH,), _f32),
            pltpu.VMEM_SHARED((PAD_HALF, KD), _f32),
            pltpu.SemaphoreType.DMA,
        ],
    )
    def k3(xs_h, keep_h, rloc_h, cglb_h, out_h,
           xc_v, sc_v, rl_v, cg_v, kp_v, acc_sh, sem):
        c = lax.axis_index("c")
        s = lax.axis_index("s")
        obase = jnp.where(c == 0, PAD_HALF, 0).astype(_i32)
        iota = lax.iota(_i32, 16)
        zero16 = jnp.zeros((16,), _f32)

        def zb(j, _):
            sc_v[j, pl.ds(0, 16)] = zero16
            sc_v[j, pl.ds(16, 16)] = zero16
            sc_v[j, pl.ds(32, 16)] = zero16
            sc_v[j, pl.ds(48, 16)] = zero16
            return 0

        lax.fori_loop(0, CH, zb, 0)
        for i in range(ACC_STRIPE // CH):
            pltpu.sync_copy(sc_v, acc_sh.at[pl.ds(s * ACC_STRIPE + i * CH, CH)])
        rem = ACC_STRIPE % CH
        if rem:
            pltpu.sync_copy(
                sc_v.at[pl.ds(0, rem)],
                acc_sh.at[pl.ds(s * ACC_STRIPE + (ACC_STRIPE // CH) * CH, rem)])
        plsc.subcore_barrier()

        base = s * (K3_CH * CH)

        def chunk(co, _):
            eoff = base + co * CH
            pltpu.sync_copy(rloc_h.at[c, pl.ds(eoff, CH)], rl_v)
            pltpu.sync_copy(cglb_h.at[c, pl.ds(eoff, CH)], cg_v)
            pltpu.sync_copy(keep_h.at[c, pl.ds(eoff, CH)], kp_v)
            cp = pltpu.async_copy(xs_h.at[cg_v], xc_v, sem)
            cp.wait()
            for g in range(CH // 16):
                wg = kp_v[pl.ds(g * 16, 16)]
                rowi = g * 16 + iota

                def scale_k(k, _):
                    ck = jnp.full((16,), 0, _i32) + k
                    vx = plsc.load_gather(xc_v, [rowi, ck])
                    plsc.store_scatter(sc_v, [rowi, ck], vx * wg)
                    return 0

                lax.fori_loop(0, KD, scale_k, 0)
            pltpu.sync_copy(sc_v, acc_sh.at[rl_v], add=True)
            return 0

        lax.fori_loop(0, K3_CH, chunk, 0)
        plsc.subcore_barrier()
        pltpu.sync_copy(
            acc_sh.at[pl.ds(s * ACC_STRIPE, ACC_STRIPE)],
            out_h.at[pl.ds(obase + s * ACC_STRIPE, ACC_STRIPE)])

    return k3(xs, keep, rowloc, colglb)


def kernel(Gu, Gi, edge_features, Wu, bu, Wi, bi, L0, rows, cols):
    r_item = rows[:EH]
    u_col = cols[:EH]
    item_pg = r_item + (PAD_HALF - NU)
    b_idx = r_item - NU

    def pad1(a, v, dt):
        return jnp.concatenate([a.astype(dt),
                                jnp.full((EP - EH,), v, dt)])

    r1 = pad1(item_pg, PAD_HALF, _i32)
    bix = pad1(b_idx, 0, _i32)
    cu = pad1(u_col, 0, _i32)
    l0u = pad1(L0[:EH], 0.0, _f32)
    l0i = pad1(L0[EH:], 0.0, _f32)
    ef_p = jnp.zeros((EP, 16), _f32).at[:EH].set(edge_features)

    rowloc = jnp.stack([bix, cu])
    colglb = jnp.stack([cu, r1])

    x = jnp.zeros((XP, KD), _f32).at[0:NU].set(Gu) \
        .at[PAD_HALF:PAD_HALF + NU].set(Gi)

    p2u, p2i = _p2_tc(ef_p, Wu, bu, Wi, bi)

    for _ in range(2):
        keep, degp = _k1_sc(x, p2u, p2i, r1, bix, cu, l0u, l0i)
        dis2d = _dis_tc(degp)
        xs = _rowscale_tc(x, dis2d)
        raw = _k3_sc(xs, keep, rowloc, colglb)
        x = _rowscale_tc(raw, dis2d)

    return x[0:NU], x[PAD_HALF:PAD_HALF + NU]


# speedup vs baseline: 7.2041x; 2.0912x over previous
import functools

import jax
import jax.numpy as jnp
from jax import lax
from jax.experimental import pallas as pl
from jax.experimental.pallas import tpu as pltpu
from jax.experimental.pallas import tpu_sc as plsc

NU = 25000
PAD_HALF = 25600
XP = 2 * PAD_HALF
KD = 64
EH = 400000
EP = 401408
NCHUNK = EP // 128
CH = 128
K1_CH = EP // (32 * CH)
K3_CH = EP // (16 * CH)
ACC_STRIPE = PAD_HALF // 16
DUMP = PAD_HALF - 1

_f32 = jnp.float32
_i32 = jnp.int32


def _p2_tc(ef_p, Wu, bu, Wi, bi):
    nblk = EP // 2048

    def body(ef_ref, wu_ref, bu_ref, wi_ref, bi_ref, pu_ref, pi_ref):
        e = ef_ref[...]
        pu = jnp.dot(e, wu_ref[...], preferred_element_type=_f32) + bu_ref[...]
        pi = jnp.dot(e, wi_ref[...], preferred_element_type=_f32) + bi_ref[...]
        pu_ref[...] = pu * pu
        pi_ref[...] = pi * pi

    return pl.pallas_call(
        body,
        grid=(nblk,),
        in_specs=[
            pl.BlockSpec((2048, 16), lambda i: (i, 0)),
            pl.BlockSpec((16, KD), lambda i: (0, 0)),
            pl.BlockSpec((1, KD), lambda i: (0, 0)),
            pl.BlockSpec((16, KD), lambda i: (0, 0)),
            pl.BlockSpec((1, KD), lambda i: (0, 0)),
        ],
        out_specs=[pl.BlockSpec((2048, KD), lambda i: (i, 0))] * 2,
        out_shape=[jax.ShapeDtypeStruct((EP, KD), _f32)] * 2,
    )(ef_p, Wu, bu.reshape(1, KD), Wi, bi.reshape(1, KD))


def _dis_tc(degp):
    def body(d_ref, o_ref):
        d = jnp.sum(d_ref[...], axis=0)
        o_ref[...] = jnp.where(d > 0, 1.0 / jnp.sqrt(d), 0.0)

    out = pl.pallas_call(
        body,
        out_shape=jax.ShapeDtypeStruct((XP // 128, 128), _f32),
    )(degp.reshape(32, XP // 128, 128))
    return out.reshape(XP, 1)


def _rowscale_tc(x, dis2d):
    nblk = XP // 2048

    def body(x_ref, d_ref, o_ref):
        o_ref[...] = x_ref[...] * d_ref[...]

    return pl.pallas_call(
        body,
        grid=(nblk,),
        in_specs=[
            pl.BlockSpec((2048, KD), lambda i: (i, 0)),
            pl.BlockSpec((2048, 1), lambda i: (i, 0)),
        ],
        out_specs=pl.BlockSpec((2048, KD), lambda i: (i, 0)),
        out_shape=jax.ShapeDtypeStruct((XP, KD), _f32),
    )(x, dis2d)


def _k1_sc(x, p2u, p2i, ipack, l0pack):
    mesh = plsc.VectorSubcoreMesh(core_axis_name="c", subcore_axis_name="s")

    @functools.partial(
        pl.kernel,
        out_type=[
            jax.ShapeDtypeStruct((2, EP), _i32),
            jax.ShapeDtypeStruct((32, XP), _f32),
        ],
        mesh=mesh,
        compiler_params=pltpu.CompilerParams(
            needs_layout_passes=False, use_tc_tiling_on_sc=False),
        scratch_types=[
            pltpu.VMEM((2, 384), _i32),
            pltpu.VMEM((2, 256), _f32),
            pltpu.VMEM((CH, KD), _f32),
            pltpu.VMEM((CH, KD), _f32),
            pltpu.VMEM((CH, KD), _f32),
            pltpu.VMEM((CH, KD), _f32),
            pltpu.VMEM((CH, KD), _f32),
            pltpu.VMEM((CH, KD), _f32),
            pltpu.VMEM((CH, KD), _f32),
            pltpu.VMEM((CH, KD), _f32),
            pltpu.VMEM((2, CH), _i32),
            pltpu.VMEM((2, CH), _i32),
            pltpu.VMEM((XP,), _f32),
            pltpu.SemaphoreType.DMA,
            pltpu.SemaphoreType.DMA,
            pltpu.SemaphoreType.DMA,
            pltpu.SemaphoreType.DMA,
            pltpu.SemaphoreType.DMA,
            pltpu.SemaphoreType.DMA,
        ],
    )
    def k1(x_h, pu_h, pi_h, ipk_h, l0_h,
           rsel_h, degp_h,
           ipk_v, l0_v, a0, a1, b0, b1, u0, u1, i0, i1,
           ru_v, ri_v, deg_v,
           sl0, sl1, sg0, sg1, so0, so1):
        c = lax.axis_index("c")
        s = lax.axis_index("s")
        wid = c * 16 + s
        iota = lax.iota(_i32, 16)
        zero16 = jnp.zeros((16,), _f32)
        a_v = (a0, a1)
        b_v = (b0, b1)
        pu_v = (u0, u1)
        pi_v = (i0, i1)
        sl = (sl0, sl1)
        sg = (sg0, sg1)
        so = (so0, so1)

        def zdeg(i, _):
            deg_v[pl.ds(i * 16, 16)] = zero16
            return 0

        lax.fori_loop(0, XP // 16, zdeg, 0)

        base = wid * K1_CH

        def lin_issue(n, p):
            pltpu.async_copy(ipk_h.at[base + n], ipk_v.at[p], sl[p])
            pltpu.async_copy(l0_h.at[base + n], l0_v.at[p], sl[p])

        def lin_wait(p):
            pltpu.make_async_copy(ipk_h.at[0], ipk_v.at[p], sl[p]).wait()
            pltpu.make_async_copy(l0_h.at[0], l0_v.at[p], sl[p]).wait()

        def gat_issue(n, p):
            eoff = (base + n) * CH
            pltpu.async_copy(x_h.at[ipk_v.at[p, pl.ds(0, CH)]], a_v[p], sg[p])
            pltpu.async_copy(x_h.at[ipk_v.at[p, pl.ds(CH, CH)]], b_v[p], sg[p])
            pltpu.async_copy(pu_h.at[pl.ds(eoff, CH)], pu_v[p], sg[p])
            pltpu.async_copy(pi_h.at[pl.ds(eoff, CH)], pi_v[p], sg[p])

        def gat_wait(p):
            pltpu.make_async_copy(x_h.at[ipk_v.at[p, pl.ds(0, CH)]], a_v[p], sg[p]).wait()
            pltpu.make_async_copy(x_h.at[ipk_v.at[p, pl.ds(CH, CH)]], b_v[p], sg[p]).wait()
            pltpu.make_async_copy(pu_h.at[pl.ds(0, CH)], pu_v[p], sg[p]).wait()
            pltpu.make_async_copy(pi_h.at[pl.ds(0, CH)], pi_v[p], sg[p]).wait()

        def out_issue(n, p):
            eoff = (base + n) * CH
            pltpu.async_copy(ru_v.at[p], rsel_h.at[0, pl.ds(eoff, CH)], so[p])
            pltpu.async_copy(ri_v.at[p], rsel_h.at[1, pl.ds(eoff, CH)], so[p])

        def out_wait(p):
            pltpu.make_async_copy(ru_v.at[p], rsel_h.at[0, pl.ds(0, CH)], so[p]).wait()
            pltpu.make_async_copy(ri_v.at[p], rsel_h.at[1, pl.ds(0, CH)], so[p]).wait()

        def compute(n, p):
            eoff = (base + n) * CH
            for g in range(CH // 16):
                rowi = g * 16 + iota

                def dot_k(t, acc):
                    abu, aau, bbu, abi, aai, bbi = acc
                    for dk in range(4):
                        ck = jnp.full((16,), 0, _i32) + (t * 4 + dk)
                        va = plsc.load_gather(a_v[p], [rowi, ck])
                        vb = plsc.load_gather(b_v[p], [rowi, ck])
                        vu = plsc.load_gather(pu_v[p], [rowi, ck])
                        vi = plsc.load_gather(pi_v[p], [rowi, ck])
                        ab = va * vb
                        aa = va * va
                        bb = vb * vb
                        abu = abu + ab * vu
                        aau = aau + aa * vu
                        bbu = bbu + bb * vu
                        abi = abi + ab * vi
                        aai = aai + aa * vi
                        bbi = bbi + bb * vi
                    return (abu, aau, bbu, abi, aai, bbi)

                z = jnp.zeros((16,), _f32)
                abu, aau, bbu, abi, aai, bbi = lax.fori_loop(
                    0, KD // 4, dot_k, (z, z, z, z, z, z))
                tl_u = 0.2 - l0_v[p, pl.ds(g * 16, 16)]
                tl_i = 0.2 - l0_v[p, pl.ds(CH + g * 16, 16)]
                e2 = jnp.float32(1e-16)
                ku = ((tl_u <= 0)
                      | ((abu > 0)
                         & (abu * abu >= tl_u * tl_u * jnp.maximum(aau, e2)
                            * jnp.maximum(bbu, e2))))
                ki = ((tl_i <= 0)
                      | ((abi > 0)
                         & (abi * abi >= tl_i * tl_i * jnp.maximum(aai, e2)
                            * jnp.maximum(bbi, e2))))
                valid = (eoff + g * 16 + iota) < EH
                ku = ku & valid
                ki = ki & valid
                bi_g = ipk_v[p, pl.ds(CH + g * 16, 16)]
                ci_g = ipk_v[p, pl.ds(2 * CH + g * 16, 16)]
                ri_g = ipk_v[p, pl.ds(g * 16, 16)]
                ru_v[p, pl.ds(g * 16, 16)] = jnp.where(ku, bi_g, DUMP)
                ri_v[p, pl.ds(g * 16, 16)] = jnp.where(ki, ci_g, DUMP)
                kuf = jnp.where(ku, 1.0, 0.0).astype(_f32)
                kif = jnp.where(ki, 1.0, 0.0).astype(_f32)
                plsc.addupdate_scatter(deg_v, [ri_g], kuf)
                plsc.addupdate_scatter(deg_v, [ci_g], kif)

        lin_issue(0, 0)
        lin_issue(1, 1)
        lin_wait(0)
        gat_issue(0, 0)

        def pair(m, _):
            for ph in range(2):
                n = m * 2 + ph
                p = ph
                q = 1 - ph
                gat_wait(p)

                @pl.when(n + 1 < K1_CH)
                def _():
                    lin_wait(q)
                    gat_issue(n + 1, q)

                @pl.when(n >= 2)
                def _():
                    out_wait(p)

                compute(n, p)
                out_issue(n, p)

                @pl.when(n + 2 < K1_CH)
                def _():
                    lin_issue(n + 2, p)
            return 0

        lax.fori_loop(0, K1_CH // 2, pair, 0)
        out_wait(0)
        out_wait(1)
        pltpu.sync_copy(deg_v, degp_h.at[wid])

    return k1(x, p2u, p2i, ipack, l0pack)


def _k3_sc(xs, rsel, colglb):
    mesh = plsc.VectorSubcoreMesh(core_axis_name="c", subcore_axis_name="s")

    @functools.partial(
        pl.kernel,
        out_type=jax.ShapeDtypeStruct((XP, KD), _f32),
        mesh=mesh,
        compiler_params=pltpu.CompilerParams(
            needs_layout_passes=False, use_tc_tiling_on_sc=False),
        scratch_types=[
            pltpu.VMEM((CH, KD), _f32),
            pltpu.VMEM((CH, KD), _f32),
            pltpu.VMEM((2, CH), _i32),
            pltpu.VMEM((2, CH), _i32),
            pltpu.VMEM((2, CH), _i32),
            pltpu.VMEM_SHARED((PAD_HALF, KD), _f32),
            pltpu.SemaphoreType.DMA,
            pltpu.SemaphoreType.DMA,
            pltpu.SemaphoreType.DMA,
            pltpu.SemaphoreType.DMA,
            pltpu.SemaphoreType.DMA,
            pltpu.SemaphoreType.DMA,
        ],
    )
    def k3(xs_h, rsel_h, cglb_h, out_h,
           xc0, xc1, rl_v, cg_v, rs_v, acc_sh,
           sl0, sl1, sg0, sg1, ss0, ss1):
        c = lax.axis_index("c")
        s = lax.axis_index("s")
        obase = jnp.where(c == 0, PAD_HALF, 0).astype(_i32)
        zero16 = jnp.zeros((16,), _f32)
        xc = (xc0, xc1)
        sl = (sl0, sl1)
        sg = (sg0, sg1)
        ss = (ss0, ss1)

        def zb(j, _):
            xc0[j, pl.ds(0, 16)] = zero16
            xc0[j, pl.ds(16, 16)] = zero16
            xc0[j, pl.ds(32, 16)] = zero16
            xc0[j, pl.ds(48, 16)] = zero16
            return 0

        lax.fori_loop(0, CH, zb, 0)
        for i in range(ACC_STRIPE // CH):
            pltpu.sync_copy(xc0, acc_sh.at[pl.ds(s * ACC_STRIPE + i * CH, CH)])
        rem = ACC_STRIPE % CH
        if rem:
            pltpu.sync_copy(
                xc0.at[pl.ds(0, rem)],
                acc_sh.at[pl.ds(s * ACC_STRIPE + (ACC_STRIPE // CH) * CH, rem)])
        plsc.subcore_barrier()

        base = s * K3_CH

        def lin_issue(n, p):
            eoff = (base + n) * CH
            pltpu.async_copy(rsel_h.at[c, pl.ds(eoff, CH)], rl_v.at[p], sl[p])
            pltpu.async_copy(cglb_h.at[c, pl.ds(eoff, CH)], cg_v.at[p], sl[p])

        def lin_wait(p):
            pltpu.make_async_copy(rsel_h.at[0, pl.ds(0, CH)], rl_v.at[p], sl[p]).wait()
            pltpu.make_async_copy(cglb_h.at[0, pl.ds(0, CH)], cg_v.at[p], sl[p]).wait()

        def gat_issue(p):
            pltpu.async_copy(xs_h.at[cg_v.at[p]], xc[p], sg[p])

        def gat_wait(p):
            pltpu.make_async_copy(xs_h.at[cg_v.at[p]], xc[p], sg[p]).wait()

        def sc_issue(p):
            pltpu.async_copy(xc[p], acc_sh.at[rs_v.at[p]], ss[p], add=True)

        def sc_wait(p):
            pltpu.make_async_copy(xc[p], acc_sh.at[rs_v.at[p]], ss[p]).wait()

        lin_issue(0, 0)
        lin_issue(1, 1)
        lin_wait(0)
        gat_issue(0)

        def pair(m, _):
            for ph in range(2):
                n = m * 2 + ph
                p = ph
                q = 1 - ph
                gat_wait(p)
                for g in range(CH // 16):
                    rs_v[p, pl.ds(g * 16, 16)] = rl_v[p, pl.ds(g * 16, 16)]
                sc_issue(p)

                @pl.when(n + 1 < K3_CH)
                def _():
                    lin_wait(q)

                @pl.when(n >= 1)
                def _():
                    sc_wait(q)

                @pl.when(n + 1 < K3_CH)
                def _():
                    gat_issue(q)

                @pl.when(n + 2 < K3_CH)
                def _():
                    lin_issue(n + 2, p)
            return 0

        lax.fori_loop(0, K3_CH // 2, pair, 0)
        sc_wait(1)
        plsc.subcore_barrier()
        pltpu.sync_copy(
            acc_sh.at[pl.ds(s * ACC_STRIPE, ACC_STRIPE)],
            out_h.at[pl.ds(obase + s * ACC_STRIPE, ACC_STRIPE)])

    return k3(xs, rsel, colglb)


def kernel(Gu, Gi, edge_features, Wu, bu, Wi, bi, L0, rows, cols):
    r_item = rows[:EH]
    u_col = cols[:EH]
    item_pg = r_item + (PAD_HALF - NU)
    b_idx = r_item - NU

    def pad1(a, v, dt):
        return jnp.concatenate([a.astype(dt),
                                jnp.full((EP - EH,), v, dt)])

    r1 = pad1(item_pg, PAD_HALF, _i32)
    bix = pad1(b_idx, 0, _i32)
    cu = pad1(u_col, 0, _i32)
    l0u = pad1(L0[:EH], 0.0, _f32)
    l0i = pad1(L0[EH:], 0.0, _f32)
    ef_p = jnp.zeros((EP, 16), _f32).at[:EH].set(edge_features)

    ipack = jnp.concatenate(
        [r1.reshape(NCHUNK, CH), bix.reshape(NCHUNK, CH),
         cu.reshape(NCHUNK, CH)], axis=1)
    l0pack = jnp.concatenate(
        [l0u.reshape(NCHUNK, CH), l0i.reshape(NCHUNK, CH)], axis=1)
    colglb = jnp.stack([cu, r1])

    x = jnp.zeros((XP, KD), _f32).at[0:NU].set(Gu) \
        .at[PAD_HALF:PAD_HALF + NU].set(Gi)

    p2u, p2i = _p2_tc(ef_p, Wu, bu, Wi, bi)

    for _ in range(2):
        rsel, degp = _k1_sc(x, p2u, p2i, ipack, l0pack)
        dis2d = _dis_tc(degp)
        xs = _rowscale_tc(x, dis2d)
        raw = _k3_sc(xs, rsel, colglb)
        x = _rowscale_tc(raw, dis2d)

    return x[0:NU], x[PAD_HALF:PAD_HALF + NU]


# speedup vs baseline: 10.6046x; 1.4720x over previous
import functools

import jax
import jax.numpy as jnp
from jax import lax
from jax.experimental import pallas as pl
from jax.experimental.pallas import tpu as pltpu
from jax.experimental.pallas import tpu_sc as plsc

NU = 25000
PAD_HALF = 25600
XP = 2 * PAD_HALF
KD = 64
EH = 400000
EP = 401408
NCHUNK = EP // 128
CH = 128
K1_CH = EP // (32 * CH)
K3_CH = EP // (16 * CH)
ACC_STRIPE = PAD_HALF // 16
DUMP = PAD_HALF - 1

_f32 = jnp.float32
_i32 = jnp.int32


def _p2_tc(efT, Wu, bu, Wi, bi):
    nblk = EP // 2048

    def body(ef_ref, wu_ref, bu_ref, wi_ref, bi_ref, pu_ref, pi_ref):
        e = ef_ref[...]
        pu = jnp.dot(wu_ref[...], e, preferred_element_type=_f32) + bu_ref[...]
        pi = jnp.dot(wi_ref[...], e, preferred_element_type=_f32) + bi_ref[...]
        pu_ref[...] = pu * pu
        pi_ref[...] = pi * pi

    return pl.pallas_call(
        body,
        grid=(nblk,),
        in_specs=[
            pl.BlockSpec((16, 2048), lambda i: (0, i)),
            pl.BlockSpec((KD, 16), lambda i: (0, 0)),
            pl.BlockSpec((KD, 1), lambda i: (0, 0)),
            pl.BlockSpec((KD, 16), lambda i: (0, 0)),
            pl.BlockSpec((KD, 1), lambda i: (0, 0)),
        ],
        out_specs=[pl.BlockSpec((KD, 2048), lambda i: (0, i))] * 2,
        out_shape=[jax.ShapeDtypeStruct((KD, EP), _f32)] * 2,
    )(efT, Wu.T, bu.reshape(KD, 1), Wi.T, bi.reshape(KD, 1))


def _dis_tc(degp):
    def body(d_ref, o_ref):
        d = jnp.sum(d_ref[...], axis=0)
        o_ref[...] = jnp.where(d > 0, 1.0 / jnp.sqrt(d), 0.0)

    out = pl.pallas_call(
        body,
        out_shape=jax.ShapeDtypeStruct((XP // 128, 128), _f32),
    )(degp.reshape(32, XP // 128, 128))
    return out.reshape(XP, 1)


def _rowscale_tc(x, dis2d):
    nblk = XP // 2048

    def body(x_ref, d_ref, o_ref):
        o_ref[...] = x_ref[...] * d_ref[...]

    return pl.pallas_call(
        body,
        grid=(nblk,),
        in_specs=[
            pl.BlockSpec((2048, KD), lambda i: (i, 0)),
            pl.BlockSpec((2048, 1), lambda i: (i, 0)),
        ],
        out_specs=pl.BlockSpec((2048, KD), lambda i: (i, 0)),
        out_shape=jax.ShapeDtypeStruct((XP, KD), _f32),
    )(x, dis2d)


def _k1_sc(x, p2u, p2i, ipack, l0pack):
    mesh = plsc.VectorSubcoreMesh(core_axis_name="c", subcore_axis_name="s")

    @functools.partial(
        pl.kernel,
        out_type=[
            jax.ShapeDtypeStruct((2, EP), _i32),
            jax.ShapeDtypeStruct((32, XP), _f32),
        ],
        mesh=mesh,
        compiler_params=pltpu.CompilerParams(
            needs_layout_passes=False, use_tc_tiling_on_sc=False),
        scratch_types=[
            pltpu.VMEM((2, 384), _i32),
            pltpu.VMEM((2, 256), _f32),
            pltpu.VMEM((CH, KD), _f32),
            pltpu.VMEM((CH, KD), _f32),
            pltpu.VMEM((CH, KD), _f32),
            pltpu.VMEM((CH, KD), _f32),
            pltpu.VMEM((KD, CH), _f32),
            pltpu.VMEM((KD, CH), _f32),
            pltpu.VMEM((KD, CH), _f32),
            pltpu.VMEM((KD, CH), _f32),
            pltpu.VMEM((2, CH), _i32),
            pltpu.VMEM((2, CH), _i32),
            pltpu.VMEM((XP,), _f32),
            pltpu.SemaphoreType.DMA,
            pltpu.SemaphoreType.DMA,
            pltpu.SemaphoreType.DMA,
            pltpu.SemaphoreType.DMA,
            pltpu.SemaphoreType.DMA,
            pltpu.SemaphoreType.DMA,
        ],
    )
    def k1(x_h, pu_h, pi_h, ipk_h, l0_h,
           rsel_h, degp_h,
           ipk_v, l0_v, a0, a1, b0, b1, u0, u1, i0, i1,
           ru_v, ri_v, deg_v,
           sl0, sl1, sg0, sg1, so0, so1):
        c = lax.axis_index("c")
        s = lax.axis_index("s")
        wid = c * 16 + s
        iota = lax.iota(_i32, 16)
        zero16 = jnp.zeros((16,), _f32)
        a_v = (a0, a1)
        b_v = (b0, b1)
        pu_v = (u0, u1)
        pi_v = (i0, i1)
        sl = (sl0, sl1)
        sg = (sg0, sg1)
        so = (so0, so1)

        def zdeg(i, _):
            deg_v[pl.ds(i * 16, 16)] = zero16
            return 0

        lax.fori_loop(0, XP // 16, zdeg, 0)

        base = wid * K1_CH

        def lin_issue(n, p):
            pltpu.async_copy(ipk_h.at[base + n], ipk_v.at[p], sl[p])
            pltpu.async_copy(l0_h.at[base + n], l0_v.at[p], sl[p])

        def lin_wait(p):
            pltpu.make_async_copy(ipk_h.at[0], ipk_v.at[p], sl[p]).wait()
            pltpu.make_async_copy(l0_h.at[0], l0_v.at[p], sl[p]).wait()

        def gat_issue(n, p):
            eoff = (base + n) * CH
            pltpu.async_copy(x_h.at[ipk_v.at[p, pl.ds(0, CH)]], a_v[p], sg[p])
            pltpu.async_copy(x_h.at[ipk_v.at[p, pl.ds(CH, CH)]], b_v[p], sg[p])
            pltpu.async_copy(pu_h.at[:, pl.ds(eoff, CH)], pu_v[p], sg[p])
            pltpu.async_copy(pi_h.at[:, pl.ds(eoff, CH)], pi_v[p], sg[p])

        def gat_wait(p):
            pltpu.make_async_copy(x_h.at[ipk_v.at[p, pl.ds(0, CH)]], a_v[p], sg[p]).wait()
            pltpu.make_async_copy(x_h.at[ipk_v.at[p, pl.ds(CH, CH)]], b_v[p], sg[p]).wait()
            pltpu.make_async_copy(pu_h.at[:, pl.ds(0, CH)], pu_v[p], sg[p]).wait()
            pltpu.make_async_copy(pi_h.at[:, pl.ds(0, CH)], pi_v[p], sg[p]).wait()

        def out_issue(n, p):
            eoff = (base + n) * CH
            pltpu.async_copy(ru_v.at[p], rsel_h.at[0, pl.ds(eoff, CH)], so[p])
            pltpu.async_copy(ri_v.at[p], rsel_h.at[1, pl.ds(eoff, CH)], so[p])

        def out_wait(p):
            pltpu.make_async_copy(ru_v.at[p], rsel_h.at[0, pl.ds(0, CH)], so[p]).wait()
            pltpu.make_async_copy(ri_v.at[p], rsel_h.at[1, pl.ds(0, CH)], so[p]).wait()

        def compute(n, p):
            eoff = (base + n) * CH
            for half in range(2):
                gset = [half * 4 + gg for gg in range(4)]
                rowis = [g * 16 + iota for g in gset]

                def dot_k(k, acc):
                    acc = list(acc)
                    ck = jnp.zeros((16,), _i32) + k
                    va = [plsc.load_gather(a_v[p], [rowis[gg], ck])
                          for gg in range(4)]
                    vb = [plsc.load_gather(b_v[p], [rowis[gg], ck])
                          for gg in range(4)]
                    vu = [pu_v[p][k, pl.ds(g * 16, 16)] for g in gset]
                    vi = [pi_v[p][k, pl.ds(g * 16, 16)] for g in gset]
                    for gg in range(4):
                        ab = va[gg] * vb[gg]
                        aa = va[gg] * va[gg]
                        bb = vb[gg] * vb[gg]
                        o = gg * 6
                        acc[o + 0] = acc[o + 0] + ab * vu[gg]
                        acc[o + 1] = acc[o + 1] + aa * vu[gg]
                        acc[o + 2] = acc[o + 2] + bb * vu[gg]
                        acc[o + 3] = acc[o + 3] + ab * vi[gg]
                        acc[o + 4] = acc[o + 4] + aa * vi[gg]
                        acc[o + 5] = acc[o + 5] + bb * vi[gg]
                    return tuple(acc)

                z = jnp.zeros((16,), _f32)
                accs = lax.fori_loop(0, KD, dot_k, (z,) * 24)
                for gg in range(4):
                    g = gset[gg]
                    abu, aau, bbu, abi, aai, bbi = accs[gg * 6:gg * 6 + 6]
                    tl_u = 0.2 - l0_v[p, pl.ds(g * 16, 16)]
                    tl_i = 0.2 - l0_v[p, pl.ds(CH + g * 16, 16)]
                    e2 = jnp.float32(1e-16)
                    ku = ((tl_u <= 0)
                          | ((abu > 0)
                             & (abu * abu >= tl_u * tl_u * jnp.maximum(aau, e2)
                                * jnp.maximum(bbu, e2))))
                    ki = ((tl_i <= 0)
                          | ((abi > 0)
                             & (abi * abi >= tl_i * tl_i * jnp.maximum(aai, e2)
                                * jnp.maximum(bbi, e2))))
                    valid = (eoff + g * 16 + iota) < EH
                    ku = ku & valid
                    ki = ki & valid
                    bi_g = ipk_v[p, pl.ds(CH + g * 16, 16)]
                    ci_g = ipk_v[p, pl.ds(2 * CH + g * 16, 16)]
                    ri_g = ipk_v[p, pl.ds(g * 16, 16)]
                    ru_v[p, pl.ds(g * 16, 16)] = jnp.where(ku, bi_g, DUMP)
                    ri_v[p, pl.ds(g * 16, 16)] = jnp.where(ki, ci_g, DUMP)
                    kuf = jnp.where(ku, 1.0, 0.0).astype(_f32)
                    kif = jnp.where(ki, 1.0, 0.0).astype(_f32)
                    plsc.addupdate_scatter(deg_v, [ri_g], kuf)
                    plsc.addupdate_scatter(deg_v, [ci_g], kif)

        lin_issue(0, 0)
        lin_issue(1, 1)
        lin_wait(0)
        gat_issue(0, 0)

        def pair(m, _):
            for ph in range(2):
                n = m * 2 + ph
                p = ph
                q = 1 - ph
                gat_wait(p)

                @pl.when(n + 1 < K1_CH)
                def _():
                    lin_wait(q)
                    gat_issue(n + 1, q)

                @pl.when(n >= 2)
                def _():
                    out_wait(p)

                compute(n, p)
                out_issue(n, p)

                @pl.when(n + 2 < K1_CH)
                def _():
                    lin_issue(n + 2, p)
            return 0

        lax.fori_loop(0, K1_CH // 2, pair, 0)
        out_wait(0)
        out_wait(1)
        pltpu.sync_copy(deg_v, degp_h.at[wid])

    return k1(x, p2u, p2i, ipack, l0pack)


def _k3_sc(xs, rsel, colglb):
    mesh = plsc.VectorSubcoreMesh(core_axis_name="c", subcore_axis_name="s")

    @functools.partial(
        pl.kernel,
        out_type=jax.ShapeDtypeStruct((XP, KD), _f32),
        mesh=mesh,
        compiler_params=pltpu.CompilerParams(
            needs_layout_passes=False, use_tc_tiling_on_sc=False),
        scratch_types=[
            pltpu.VMEM((CH, KD), _f32),
            pltpu.VMEM((CH, KD), _f32),
            pltpu.VMEM((2, CH), _i32),
            pltpu.VMEM((2, CH), _i32),
            pltpu.VMEM((2, CH), _i32),
            pltpu.VMEM_SHARED((PAD_HALF, KD), _f32),
            pltpu.SemaphoreType.DMA,
            pltpu.SemaphoreType.DMA,
            pltpu.SemaphoreType.DMA,
            pltpu.SemaphoreType.DMA,
            pltpu.SemaphoreType.DMA,
            pltpu.SemaphoreType.DMA,
        ],
    )
    def k3(xs_h, rsel_h, cglb_h, out_h,
           xc0, xc1, rl_v, cg_v, rs_v, acc_sh,
           sl0, sl1, sg0, sg1, ss0, ss1):
        c = lax.axis_index("c")
        s = lax.axis_index("s")
        obase = jnp.where(c == 0, PAD_HALF, 0).astype(_i32)
        zero16 = jnp.zeros((16,), _f32)
        xc = (xc0, xc1)
        sl = (sl0, sl1)
        sg = (sg0, sg1)
        ss = (ss0, ss1)

        def zb(j, _):
            xc0[j, pl.ds(0, 16)] = zero16
            xc0[j, pl.ds(16, 16)] = zero16
            xc0[j, pl.ds(32, 16)] = zero16
            xc0[j, pl.ds(48, 16)] = zero16
            return 0

        lax.fori_loop(0, CH, zb, 0)
        for i in range(ACC_STRIPE // CH):
            pltpu.sync_copy(xc0, acc_sh.at[pl.ds(s * ACC_STRIPE + i * CH, CH)])
        rem = ACC_STRIPE % CH
        if rem:
            pltpu.sync_copy(
                xc0.at[pl.ds(0, rem)],
                acc_sh.at[pl.ds(s * ACC_STRIPE + (ACC_STRIPE // CH) * CH, rem)])
        plsc.subcore_barrier()

        base = s * K3_CH

        def lin_issue(n, p):
            eoff = (base + n) * CH
            pltpu.async_copy(rsel_h.at[c, pl.ds(eoff, CH)], rl_v.at[p], sl[p])
            pltpu.async_copy(cglb_h.at[c, pl.ds(eoff, CH)], cg_v.at[p], sl[p])

        def lin_wait(p):
            pltpu.make_async_copy(rsel_h.at[0, pl.ds(0, CH)], rl_v.at[p], sl[p]).wait()
            pltpu.make_async_copy(cglb_h.at[0, pl.ds(0, CH)], cg_v.at[p], sl[p]).wait()

        def gat_issue(p):
            pltpu.async_copy(xs_h.at[cg_v.at[p]], xc[p], sg[p])

        def gat_wait(p):
            pltpu.make_async_copy(xs_h.at[cg_v.at[p]], xc[p], sg[p]).wait()

        def sc_issue(p):
            pltpu.async_copy(xc[p], acc_sh.at[rs_v.at[p]], ss[p], add=True)

        def sc_wait(p):
            pltpu.make_async_copy(xc[p], acc_sh.at[rs_v.at[p]], ss[p]).wait()

        lin_issue(0, 0)
        lin_issue(1, 1)
        lin_wait(0)
        gat_issue(0)

        def pair(m, _):
            for ph in range(2):
                n = m * 2 + ph
                p = ph
                q = 1 - ph
                gat_wait(p)
                for g in range(CH // 16):
                    rs_v[p, pl.ds(g * 16, 16)] = rl_v[p, pl.ds(g * 16, 16)]
                sc_issue(p)

                @pl.when(n + 1 < K3_CH)
                def _():
                    lin_wait(q)

                @pl.when(n >= 1)
                def _():
                    sc_wait(q)

                @pl.when(n + 1 < K3_CH)
                def _():
                    gat_issue(q)

                @pl.when(n + 2 < K3_CH)
                def _():
                    lin_issue(n + 2, p)
            return 0

        lax.fori_loop(0, K3_CH // 2, pair, 0)
        sc_wait(1)
        plsc.subcore_barrier()
        pltpu.sync_copy(
            acc_sh.at[pl.ds(s * ACC_STRIPE, ACC_STRIPE)],
            out_h.at[pl.ds(obase + s * ACC_STRIPE, ACC_STRIPE)])

    return k3(xs, rsel, colglb)


def kernel(Gu, Gi, edge_features, Wu, bu, Wi, bi, L0, rows, cols):
    r_item = rows[:EH]
    u_col = cols[:EH]
    item_pg = r_item + (PAD_HALF - NU)
    b_idx = r_item - NU

    def pad1(a, v, dt):
        return jnp.concatenate([a.astype(dt),
                                jnp.full((EP - EH,), v, dt)])

    r1 = pad1(item_pg, PAD_HALF, _i32)
    bix = pad1(b_idx, 0, _i32)
    cu = pad1(u_col, 0, _i32)
    l0u = pad1(L0[:EH], 0.0, _f32)
    l0i = pad1(L0[EH:], 0.0, _f32)
    efT = jnp.zeros((16, EP), _f32).at[:, :EH].set(edge_features.T)

    ipack = jnp.concatenate(
        [r1.reshape(NCHUNK, CH), bix.reshape(NCHUNK, CH),
         cu.reshape(NCHUNK, CH)], axis=1)
    l0pack = jnp.concatenate(
        [l0u.reshape(NCHUNK, CH), l0i.reshape(NCHUNK, CH)], axis=1)
    colglb = jnp.stack([cu, r1])

    x = jnp.zeros((XP, KD), _f32).at[0:NU].set(Gu) \
        .at[PAD_HALF:PAD_HALF + NU].set(Gi)

    p2u, p2i = _p2_tc(efT, Wu, bu, Wi, bi)

    for _ in range(2):
        rsel, degp = _k1_sc(x, p2u, p2i, ipack, l0pack)
        dis2d = _dis_tc(degp)
        xs = _rowscale_tc(x, dis2d)
        raw = _k3_sc(xs, rsel, colglb)
        x = _rowscale_tc(raw, dis2d)

    return x[0:NU], x[PAD_HALF:PAD_HALF + NU]


# speedup vs baseline: 16.4943x; 1.5554x over previous
import functools

import jax
import jax.numpy as jnp
from jax import lax
from jax.experimental import pallas as pl
from jax.experimental.pallas import tpu as pltpu
from jax.experimental.pallas import tpu_sc as plsc

NU = 25000
PAD_HALF = 25600
XP = 2 * PAD_HALF
KD = 64
EH = 400000
EP = 401408
NCHUNK = EP // 128
CH = 128
K1_CH = EP // (32 * CH)
K3_CH = EP // (16 * CH)
ACC_STRIPE = PAD_HALF // 16
DUMP = PAD_HALF - 1

_f32 = jnp.float32
_i32 = jnp.int32


def _p2_tc(ef_p, Wu, bu, Wi, bi):
    nblk = EP // 2048

    def body(ef_ref, wu_ref, bu_ref, wi_ref, bi_ref, pu_ref, pi_ref):
        e = ef_ref[...]
        pu = jnp.dot(e, wu_ref[...], preferred_element_type=_f32) + bu_ref[...]
        pi = jnp.dot(e, wi_ref[...], preferred_element_type=_f32) + bi_ref[...]
        pu_ref[...] = pu * pu
        pi_ref[...] = pi * pi

    return pl.pallas_call(
        body,
        grid=(nblk,),
        in_specs=[
            pl.BlockSpec((2048, 16), lambda i: (i, 0)),
            pl.BlockSpec((16, KD), lambda i: (0, 0)),
            pl.BlockSpec((1, KD), lambda i: (0, 0)),
            pl.BlockSpec((16, KD), lambda i: (0, 0)),
            pl.BlockSpec((1, KD), lambda i: (0, 0)),
        ],
        out_specs=[pl.BlockSpec((2048, KD), lambda i: (i, 0))] * 2,
        out_shape=[jax.ShapeDtypeStruct((EP, KD), _f32)] * 2,
    )(ef_p, Wu, bu.reshape(1, KD), Wi, bi.reshape(1, KD))


def _dis_tc(degp):
    def body(d_ref, o_ref):
        d = jnp.sum(d_ref[...], axis=0)
        o_ref[...] = jnp.where(d > 0, 1.0 / jnp.sqrt(d), 0.0)

    out = pl.pallas_call(
        body,
        out_shape=jax.ShapeDtypeStruct((XP // 128, 128), _f32),
    )(degp.reshape(32, XP // 128, 128))
    return out.reshape(XP, 1)


def _rowscale_tc(x, dis2d):
    nblk = XP // 2048

    def body(x_ref, d_ref, o_ref):
        o_ref[...] = x_ref[...] * d_ref[...]

    return pl.pallas_call(
        body,
        grid=(nblk,),
        in_specs=[
            pl.BlockSpec((2048, KD), lambda i: (i, 0)),
            pl.BlockSpec((2048, 1), lambda i: (i, 0)),
        ],
        out_specs=pl.BlockSpec((2048, KD), lambda i: (i, 0)),
        out_shape=jax.ShapeDtypeStruct((XP, KD), _f32),
    )(x, dis2d)


def _k1_sc(x, p2u, p2i, ipack, l0pack):
    mesh = plsc.VectorSubcoreMesh(core_axis_name="c", subcore_axis_name="s")

    @functools.partial(
        pl.kernel,
        out_type=[
            jax.ShapeDtypeStruct((2, EP), _i32),
            jax.ShapeDtypeStruct((32, XP), _f32),
        ],
        mesh=mesh,
        compiler_params=pltpu.CompilerParams(
            needs_layout_passes=False, use_tc_tiling_on_sc=False),
        scratch_types=[
            pltpu.VMEM((2, 384), _i32),
            pltpu.VMEM((2, 256), _f32),
            pltpu.VMEM((CH, KD), _f32),
            pltpu.VMEM((CH, KD), _f32),
            pltpu.VMEM((CH, KD), _f32),
            pltpu.VMEM((CH, KD), _f32),
            pltpu.VMEM((CH, KD), _f32),
            pltpu.VMEM((CH, KD), _f32),
            pltpu.VMEM((CH, KD), _f32),
            pltpu.VMEM((CH, KD), _f32),
            pltpu.VMEM((2, CH), _i32),
            pltpu.VMEM((2, CH), _i32),
            pltpu.VMEM((XP,), _f32),
            pltpu.SemaphoreType.DMA,
            pltpu.SemaphoreType.DMA,
            pltpu.SemaphoreType.DMA,
            pltpu.SemaphoreType.DMA,
            pltpu.SemaphoreType.DMA,
            pltpu.SemaphoreType.DMA,
        ],
    )
    def k1(x_h, pu_h, pi_h, ipk_h, l0_h,
           rsel_h, degp_h,
           ipk_v, l0_v, a0, a1, b0, b1, u0, u1, i0, i1,
           ru_v, ri_v, deg_v,
           sl0, sl1, sg0, sg1, so0, so1):
        c = lax.axis_index("c")
        s = lax.axis_index("s")
        wid = c * 16 + s
        iota = lax.iota(_i32, 16)
        zero16 = jnp.zeros((16,), _f32)
        a_v = (a0, a1)
        b_v = (b0, b1)
        pu_v = (u0, u1)
        pi_v = (i0, i1)
        sl = (sl0, sl1)
        sg = (sg0, sg1)
        so = (so0, so1)

        def zdeg(i, _):
            deg_v[pl.ds(i * 16, 16)] = zero16
            return 0

        lax.fori_loop(0, XP // 16, zdeg, 0)

        base = wid * K1_CH

        def lin_issue(n, p):
            pltpu.async_copy(ipk_h.at[base + n], ipk_v.at[p], sl[p])
            pltpu.async_copy(l0_h.at[base + n], l0_v.at[p], sl[p])

        def lin_wait(p):
            pltpu.make_async_copy(ipk_h.at[0], ipk_v.at[p], sl[p]).wait()
            pltpu.make_async_copy(l0_h.at[0], l0_v.at[p], sl[p]).wait()

        def gat_issue(n, p):
            eoff = (base + n) * CH
            pltpu.async_copy(x_h.at[ipk_v.at[p, pl.ds(0, CH)]], a_v[p], sg[p])
            pltpu.async_copy(x_h.at[ipk_v.at[p, pl.ds(CH, CH)]], b_v[p], sg[p])
            pltpu.async_copy(pu_h.at[pl.ds(eoff, CH)], pu_v[p], sg[p])
            pltpu.async_copy(pi_h.at[pl.ds(eoff, CH)], pi_v[p], sg[p])

        def gat_wait(p):
            pltpu.make_async_copy(x_h.at[ipk_v.at[p, pl.ds(0, CH)]], a_v[p], sg[p]).wait()
            pltpu.make_async_copy(x_h.at[ipk_v.at[p, pl.ds(CH, CH)]], b_v[p], sg[p]).wait()
            pltpu.make_async_copy(pu_h.at[pl.ds(0, CH)], pu_v[p], sg[p]).wait()
            pltpu.make_async_copy(pi_h.at[pl.ds(0, CH)], pi_v[p], sg[p]).wait()

        def out_issue(n, p):
            eoff = (base + n) * CH
            pltpu.async_copy(ru_v.at[p], rsel_h.at[0, pl.ds(eoff, CH)], so[p])
            pltpu.async_copy(ri_v.at[p], rsel_h.at[1, pl.ds(eoff, CH)], so[p])

        def out_wait(p):
            pltpu.make_async_copy(ru_v.at[p], rsel_h.at[0, pl.ds(0, CH)], so[p]).wait()
            pltpu.make_async_copy(ri_v.at[p], rsel_h.at[1, pl.ds(0, CH)], so[p]).wait()

        def compute(n, p):
            eoff = (base + n) * CH
            for half in range(2):
                gset = [half * 4 + gg for gg in range(4)]
                rowis = [g * 16 + iota for g in gset]

                def dot_k(k, acc):
                    acc = list(acc)
                    ck = (iota + k) & (KD - 1)
                    va = [plsc.load_gather(a_v[p], [rowis[gg], ck])
                          for gg in range(4)]
                    vb = [plsc.load_gather(b_v[p], [rowis[gg], ck])
                          for gg in range(4)]
                    vu = [plsc.load_gather(pu_v[p], [rowis[gg], ck])
                          for gg in range(4)]
                    vi = [plsc.load_gather(pi_v[p], [rowis[gg], ck])
                          for gg in range(4)]
                    for gg in range(4):
                        ab = va[gg] * vb[gg]
                        aa = va[gg] * va[gg]
                        bb = vb[gg] * vb[gg]
                        o = gg * 6
                        acc[o + 0] = acc[o + 0] + ab * vu[gg]
                        acc[o + 1] = acc[o + 1] + aa * vu[gg]
                        acc[o + 2] = acc[o + 2] + bb * vu[gg]
                        acc[o + 3] = acc[o + 3] + ab * vi[gg]
                        acc[o + 4] = acc[o + 4] + aa * vi[gg]
                        acc[o + 5] = acc[o + 5] + bb * vi[gg]
                    return tuple(acc)

                z = jnp.zeros((16,), _f32)
                accs = lax.fori_loop(0, KD, dot_k, (z,) * 24)
                for gg in range(4):
                    g = gset[gg]
                    abu, aau, bbu, abi, aai, bbi = accs[gg * 6:gg * 6 + 6]
                    tl_u = 0.2 - l0_v[p, pl.ds(g * 16, 16)]
                    tl_i = 0.2 - l0_v[p, pl.ds(CH + g * 16, 16)]
                    e2 = jnp.float32(1e-16)
                    ku = ((tl_u <= 0)
                          | ((abu > 0)
                             & (abu * abu >= tl_u * tl_u * jnp.maximum(aau, e2)
                                * jnp.maximum(bbu, e2))))
                    ki = ((tl_i <= 0)
                          | ((abi > 0)
                             & (abi * abi >= tl_i * tl_i * jnp.maximum(aai, e2)
                                * jnp.maximum(bbi, e2))))
                    valid = (eoff + g * 16 + iota) < EH
                    ku = ku & valid
                    ki = ki & valid
                    bi_g = ipk_v[p, pl.ds(CH + g * 16, 16)]
                    ci_g = ipk_v[p, pl.ds(2 * CH + g * 16, 16)]
                    ri_g = ipk_v[p, pl.ds(g * 16, 16)]
                    ru_v[p, pl.ds(g * 16, 16)] = jnp.where(ku, bi_g, DUMP)
                    ri_v[p, pl.ds(g * 16, 16)] = jnp.where(ki, ci_g, DUMP)
                    kuf = jnp.where(ku, 1.0, 0.0).astype(_f32)
                    kif = jnp.where(ki, 1.0, 0.0).astype(_f32)
                    plsc.addupdate_scatter(deg_v, [ri_g], kuf)
                    plsc.addupdate_scatter(deg_v, [ci_g], kif)

        lin_issue(0, 0)
        lin_issue(1, 1)
        lin_wait(0)
        gat_issue(0, 0)

        def pair(m, _):
            for ph in range(2):
                n = m * 2 + ph
                p = ph
                q = 1 - ph
                gat_wait(p)

                @pl.when(n + 1 < K1_CH)
                def _():
                    lin_wait(q)
                    gat_issue(n + 1, q)

                @pl.when(n >= 2)
                def _():
                    out_wait(p)

                compute(n, p)
                out_issue(n, p)

                @pl.when(n + 2 < K1_CH)
                def _():
                    lin_issue(n + 2, p)
            return 0

        lax.fori_loop(0, K1_CH // 2, pair, 0)
        out_wait(0)
        out_wait(1)
        pltpu.sync_copy(deg_v, degp_h.at[wid])

    return k1(x, p2u, p2i, ipack, l0pack)


def _k3_sc(xs, rsel, colglb):
    mesh = plsc.VectorSubcoreMesh(core_axis_name="c", subcore_axis_name="s")

    @functools.partial(
        pl.kernel,
        out_type=jax.ShapeDtypeStruct((XP, KD), _f32),
        mesh=mesh,
        compiler_params=pltpu.CompilerParams(
            needs_layout_passes=False, use_tc_tiling_on_sc=False),
        scratch_types=[
            pltpu.VMEM((CH, KD), _f32),
            pltpu.VMEM((CH, KD), _f32),
            pltpu.VMEM((2, CH), _i32),
            pltpu.VMEM((2, CH), _i32),
            pltpu.VMEM((2, CH), _i32),
            pltpu.VMEM_SHARED((PAD_HALF, KD), _f32),
            pltpu.SemaphoreType.DMA,
            pltpu.SemaphoreType.DMA,
            pltpu.SemaphoreType.DMA,
            pltpu.SemaphoreType.DMA,
            pltpu.SemaphoreType.DMA,
            pltpu.SemaphoreType.DMA,
        ],
    )
    def k3(xs_h, rsel_h, cglb_h, out_h,
           xc0, xc1, rl_v, cg_v, rs_v, acc_sh,
           sl0, sl1, sg0, sg1, ss0, ss1):
        c = lax.axis_index("c")
        s = lax.axis_index("s")
        obase = jnp.where(c == 0, PAD_HALF, 0).astype(_i32)
        zero16 = jnp.zeros((16,), _f32)
        xc = (xc0, xc1)
        sl = (sl0, sl1)
        sg = (sg0, sg1)
        ss = (ss0, ss1)

        def zb(j, _):
            xc0[j, pl.ds(0, 16)] = zero16
            xc0[j, pl.ds(16, 16)] = zero16
            xc0[j, pl.ds(32, 16)] = zero16
            xc0[j, pl.ds(48, 16)] = zero16
            return 0

        lax.fori_loop(0, CH, zb, 0)
        for i in range(ACC_STRIPE // CH):
            pltpu.sync_copy(xc0, acc_sh.at[pl.ds(s * ACC_STRIPE + i * CH, CH)])
        rem = ACC_STRIPE % CH
        if rem:
            pltpu.sync_copy(
                xc0.at[pl.ds(0, rem)],
                acc_sh.at[pl.ds(s * ACC_STRIPE + (ACC_STRIPE // CH) * CH, rem)])
        plsc.subcore_barrier()

        base = s * K3_CH

        def lin_issue(n, p):
            eoff = (base + n) * CH
            pltpu.async_copy(rsel_h.at[c, pl.ds(eoff, CH)], rl_v.at[p], sl[p])
            pltpu.async_copy(cglb_h.at[c, pl.ds(eoff, CH)], cg_v.at[p], sl[p])

        def lin_wait(p):
            pltpu.make_async_copy(rsel_h.at[0, pl.ds(0, CH)], rl_v.at[p], sl[p]).wait()
            pltpu.make_async_copy(cglb_h.at[0, pl.ds(0, CH)], cg_v.at[p], sl[p]).wait()

        def gat_issue(p):
            pltpu.async_copy(xs_h.at[cg_v.at[p]], xc[p], sg[p])

        def gat_wait(p):
            pltpu.make_async_copy(xs_h.at[cg_v.at[p]], xc[p], sg[p]).wait()

        def sc_issue(p):
            pltpu.async_copy(xc[p], acc_sh.at[rs_v.at[p]], ss[p], add=True)

        def sc_wait(p):
            pltpu.make_async_copy(xc[p], acc_sh.at[rs_v.at[p]], ss[p]).wait()

        lin_issue(0, 0)
        lin_issue(1, 1)
        lin_wait(0)
        gat_issue(0)

        def pair(m, _):
            for ph in range(2):
                n = m * 2 + ph
                p = ph
                q = 1 - ph
                gat_wait(p)
                for g in range(CH // 16):
                    rs_v[p, pl.ds(g * 16, 16)] = rl_v[p, pl.ds(g * 16, 16)]
                sc_issue(p)

                @pl.when(n + 1 < K3_CH)
                def _():
                    lin_wait(q)

                @pl.when(n >= 1)
                def _():
                    sc_wait(q)

                @pl.when(n + 1 < K3_CH)
                def _():
                    gat_issue(q)

                @pl.when(n + 2 < K3_CH)
                def _():
                    lin_issue(n + 2, p)
            return 0

        lax.fori_loop(0, K3_CH // 2, pair, 0)
        sc_wait(1)
        plsc.subcore_barrier()
        pltpu.sync_copy(
            acc_sh.at[pl.ds(s * ACC_STRIPE, ACC_STRIPE)],
            out_h.at[pl.ds(obase + s * ACC_STRIPE, ACC_STRIPE)])

    return k3(xs, rsel, colglb)


def kernel(Gu, Gi, edge_features, Wu, bu, Wi, bi, L0, rows, cols):
    r_item = rows[:EH]
    u_col = cols[:EH]
    item_pg = r_item + (PAD_HALF - NU)
    b_idx = r_item - NU

    def pad1(a, v, dt):
        return jnp.concatenate([a.astype(dt),
                                jnp.full((EP - EH,), v, dt)])

    r1 = pad1(item_pg, PAD_HALF, _i32)
    bix = pad1(b_idx, 0, _i32)
    cu = pad1(u_col, 0, _i32)
    l0u = pad1(L0[:EH], 0.0, _f32)
    l0i = pad1(L0[EH:], 0.0, _f32)
    ef_p = jnp.zeros((EP, 16), _f32).at[:EH].set(edge_features)

    ipack = jnp.concatenate(
        [r1.reshape(NCHUNK, CH), bix.reshape(NCHUNK, CH),
         cu.reshape(NCHUNK, CH)], axis=1)
    l0pack = jnp.concatenate(
        [l0u.reshape(NCHUNK, CH), l0i.reshape(NCHUNK, CH)], axis=1)
    colglb = jnp.stack([cu, r1])

    x = jnp.zeros((XP, KD), _f32).at[0:NU].set(Gu) \
        .at[PAD_HALF:PAD_HALF + NU].set(Gi)

    p2u, p2i = _p2_tc(ef_p, Wu, bu, Wi, bi)

    for _ in range(2):
        rsel, degp = _k1_sc(x, p2u, p2i, ipack, l0pack)
        dis2d = _dis_tc(degp)
        xs = _rowscale_tc(x, dis2d)
        raw = _k3_sc(xs, rsel, colglb)
        x = _rowscale_tc(raw, dis2d)

    return x[0:NU], x[PAD_HALF:PAD_HALF + NU]


# speedup vs baseline: 17.3923x; 1.0544x over previous
import functools

import jax
import jax.numpy as jnp
from jax import lax
from jax.experimental import pallas as pl
from jax.experimental.pallas import tpu as pltpu
from jax.experimental.pallas import tpu_sc as plsc

NU = 25000
PAD_HALF = 25600
XP = 2 * PAD_HALF
KD = 64
EH = 400000
EP = 401408
NCHUNK = EP // 128
CH = 128
K1_CH = EP // (32 * CH)
K3_CH = EP // (16 * CH)
ACC_STRIPE = PAD_HALF // 16
DUMP = PAD_HALF - 1

_f32 = jnp.float32
_i32 = jnp.int32


def _p2_tc(ef, Wu, bu, Wi, bi):
    nblk = EP // 2048

    def body(ef_ref, wu_ref, bu_ref, wi_ref, bi_ref, pu_ref, pi_ref):
        e = ef_ref[...]
        pu = jnp.dot(e, wu_ref[...], preferred_element_type=_f32) + bu_ref[...]
        pi = jnp.dot(e, wi_ref[...], preferred_element_type=_f32) + bi_ref[...]
        pu_ref[...] = pu * pu
        pi_ref[...] = pi * pi

    return pl.pallas_call(
        body,
        grid=(nblk,),
        in_specs=[
            pl.BlockSpec((2048, 16), lambda i: (i, 0)),
            pl.BlockSpec((16, KD), lambda i: (0, 0)),
            pl.BlockSpec((1, KD), lambda i: (0, 0)),
            pl.BlockSpec((16, KD), lambda i: (0, 0)),
            pl.BlockSpec((1, KD), lambda i: (0, 0)),
        ],
        out_specs=[pl.BlockSpec((2048, KD), lambda i: (i, 0))] * 2,
        out_shape=[jax.ShapeDtypeStruct((EP, KD), _f32)] * 2,
    )(ef, Wu, bu.reshape(1, KD), Wi, bi.reshape(1, KD))


def _final_tc(raw, dis2d):
    blk = 1600
    nblk = PAD_HALF // blk

    def body(xu_ref, du_ref, xi_ref, di_ref, ou_ref, oi_ref):
        ou_ref[...] = xu_ref[...] * du_ref[...]
        oi_ref[...] = xi_ref[...] * di_ref[...]

    return pl.pallas_call(
        body,
        grid=(nblk,),
        in_specs=[
            pl.BlockSpec((blk, KD), lambda i: (i, 0)),
            pl.BlockSpec((blk, 1), lambda i: (i, 0)),
            pl.BlockSpec((blk, KD), lambda i: (nblk + i, 0)),
            pl.BlockSpec((blk, 1), lambda i: (nblk + i, 0)),
        ],
        out_specs=[pl.BlockSpec((blk, KD), lambda i: (i, 0))] * 2,
        out_shape=[jax.ShapeDtypeStruct((NU, KD), _f32)] * 2,
    )(raw, dis2d, raw, dis2d)


def _dis_tc(degp):
    def body(d_ref, o_ref):
        d = jnp.sum(d_ref[...], axis=0)
        o_ref[...] = jnp.where(d > 0, 1.0 / jnp.sqrt(d), 0.0)

    out = pl.pallas_call(
        body,
        out_shape=jax.ShapeDtypeStruct((XP // 128, 128), _f32),
    )(degp.reshape(32, XP // 128, 128))
    return out.reshape(XP, 1)


def _rowscale_tc(x, dis2d):
    nblk = XP // 2048

    def body(x_ref, d_ref, o_ref):
        o_ref[...] = x_ref[...] * d_ref[...]

    return pl.pallas_call(
        body,
        grid=(nblk,),
        in_specs=[
            pl.BlockSpec((2048, KD), lambda i: (i, 0)),
            pl.BlockSpec((2048, 1), lambda i: (i, 0)),
        ],
        out_specs=pl.BlockSpec((2048, KD), lambda i: (i, 0)),
        out_shape=jax.ShapeDtypeStruct((XP, KD), _f32),
    )(x, dis2d)


def _k1_sc(x, p2u, p2i, ipack, l0pack):
    mesh = plsc.VectorSubcoreMesh(core_axis_name="c", subcore_axis_name="s")

    @functools.partial(
        pl.kernel,
        out_type=[
            jax.ShapeDtypeStruct((2, EP), _i32),
            jax.ShapeDtypeStruct((32, XP), _f32),
        ],
        mesh=mesh,
        compiler_params=pltpu.CompilerParams(
            needs_layout_passes=False, use_tc_tiling_on_sc=False),
        scratch_types=[
            pltpu.VMEM((2, 384), _i32),
            pltpu.VMEM((2, 256), _f32),
            pltpu.VMEM((CH, KD), _f32),
            pltpu.VMEM((CH, KD), _f32),
            pltpu.VMEM((CH, KD), _f32),
            pltpu.VMEM((CH, KD), _f32),
            pltpu.VMEM((CH, KD), _f32),
            pltpu.VMEM((CH, KD), _f32),
            pltpu.VMEM((CH, KD), _f32),
            pltpu.VMEM((CH, KD), _f32),
            pltpu.VMEM((2, CH), _i32),
            pltpu.VMEM((2, CH), _i32),
            pltpu.VMEM((XP,), _f32),
            pltpu.SemaphoreType.DMA,
            pltpu.SemaphoreType.DMA,
            pltpu.SemaphoreType.DMA,
            pltpu.SemaphoreType.DMA,
            pltpu.SemaphoreType.DMA,
            pltpu.SemaphoreType.DMA,
        ],
    )
    def k1(x_h, pu_h, pi_h, ipk_h, l0_h,
           rsel_h, degp_h,
           ipk_v, l0_v, a0, a1, b0, b1, u0, u1, i0, i1,
           ru_v, ri_v, deg_v,
           sl0, sl1, sg0, sg1, so0, so1):
        c = lax.axis_index("c")
        s = lax.axis_index("s")
        wid = c * 16 + s
        iota = lax.iota(_i32, 16)
        zero16 = jnp.zeros((16,), _f32)
        a_v = (a0, a1)
        b_v = (b0, b1)
        pu_v = (u0, u1)
        pi_v = (i0, i1)
        sl = (sl0, sl1)
        sg = (sg0, sg1)
        so = (so0, so1)

        def zdeg(i, _):
            deg_v[pl.ds(i * 16, 16)] = zero16
            return 0

        lax.fori_loop(0, XP // 16, zdeg, 0)

        base = wid * K1_CH

        def lin_issue(n, p):
            pltpu.async_copy(ipk_h.at[base + n], ipk_v.at[p], sl[p])
            pltpu.async_copy(l0_h.at[base + n], l0_v.at[p], sl[p])

        def lin_wait(p):
            pltpu.make_async_copy(ipk_h.at[0], ipk_v.at[p], sl[p]).wait()
            pltpu.make_async_copy(l0_h.at[0], l0_v.at[p], sl[p]).wait()

        def gat_issue(n, p):
            eoff = (base + n) * CH
            pltpu.async_copy(x_h.at[ipk_v.at[p, pl.ds(0, CH)]], a_v[p], sg[p])
            pltpu.async_copy(x_h.at[ipk_v.at[p, pl.ds(CH, CH)]], b_v[p], sg[p])
            pltpu.async_copy(pu_h.at[pl.ds(eoff, CH)], pu_v[p], sg[p])
            pltpu.async_copy(pi_h.at[pl.ds(eoff, CH)], pi_v[p], sg[p])

        def gat_wait(p):
            pltpu.make_async_copy(x_h.at[ipk_v.at[p, pl.ds(0, CH)]], a_v[p], sg[p]).wait()
            pltpu.make_async_copy(x_h.at[ipk_v.at[p, pl.ds(CH, CH)]], b_v[p], sg[p]).wait()
            pltpu.make_async_copy(pu_h.at[pl.ds(0, CH)], pu_v[p], sg[p]).wait()
            pltpu.make_async_copy(pi_h.at[pl.ds(0, CH)], pi_v[p], sg[p]).wait()

        def out_issue(n, p):
            eoff = (base + n) * CH
            pltpu.async_copy(ru_v.at[p], rsel_h.at[0, pl.ds(eoff, CH)], so[p])
            pltpu.async_copy(ri_v.at[p], rsel_h.at[1, pl.ds(eoff, CH)], so[p])

        def out_wait(p):
            pltpu.make_async_copy(ru_v.at[p], rsel_h.at[0, pl.ds(0, CH)], so[p]).wait()
            pltpu.make_async_copy(ri_v.at[p], rsel_h.at[1, pl.ds(0, CH)], so[p]).wait()

        def compute(n, p):
            eoff = (base + n) * CH
            for half in range(2):
                gset = [half * 4 + gg for gg in range(4)]
                rowis = [g * 16 + iota for g in gset]

                def dot_k(k, acc):
                    acc = list(acc)
                    ck = (iota + k) & (KD - 1)
                    va = [plsc.load_gather(a_v[p], [rowis[gg], ck])
                          for gg in range(4)]
                    vb = [plsc.load_gather(b_v[p], [rowis[gg], ck])
                          for gg in range(4)]
                    vu = [plsc.load_gather(pu_v[p], [rowis[gg], ck])
                          for gg in range(4)]
                    vi = [plsc.load_gather(pi_v[p], [rowis[gg], ck])
                          for gg in range(4)]
                    for gg in range(4):
                        ab = va[gg] * vb[gg]
                        aa = va[gg] * va[gg]
                        bb = vb[gg] * vb[gg]
                        o = gg * 6
                        acc[o + 0] = acc[o + 0] + ab * vu[gg]
                        acc[o + 1] = acc[o + 1] + aa * vu[gg]
                        acc[o + 2] = acc[o + 2] + bb * vu[gg]
                        acc[o + 3] = acc[o + 3] + ab * vi[gg]
                        acc[o + 4] = acc[o + 4] + aa * vi[gg]
                        acc[o + 5] = acc[o + 5] + bb * vi[gg]
                    return tuple(acc)

                z = jnp.zeros((16,), _f32)
                accs = lax.fori_loop(0, KD, dot_k, (z,) * 24)
                for gg in range(4):
                    g = gset[gg]
                    abu, aau, bbu, abi, aai, bbi = accs[gg * 6:gg * 6 + 6]
                    tl_u = 0.2 - l0_v[p, pl.ds(g * 16, 16)]
                    tl_i = 0.2 - l0_v[p, pl.ds(CH + g * 16, 16)]
                    e2 = jnp.float32(1e-16)
                    ku = ((tl_u <= 0)
                          | ((abu > 0)
                             & (abu * abu >= tl_u * tl_u * jnp.maximum(aau, e2)
                                * jnp.maximum(bbu, e2))))
                    ki = ((tl_i <= 0)
                          | ((abi > 0)
                             & (abi * abi >= tl_i * tl_i * jnp.maximum(aai, e2)
                                * jnp.maximum(bbi, e2))))
                    valid = (eoff + g * 16 + iota) < EH
                    ku = ku & valid
                    ki = ki & valid
                    bi_g = ipk_v[p, pl.ds(CH + g * 16, 16)]
                    ci_g = ipk_v[p, pl.ds(2 * CH + g * 16, 16)]
                    ri_g = ipk_v[p, pl.ds(g * 16, 16)]
                    ru_v[p, pl.ds(g * 16, 16)] = jnp.where(ku, bi_g, DUMP)
                    ri_v[p, pl.ds(g * 16, 16)] = jnp.where(ki, ci_g, DUMP)
                    kuf = jnp.where(ku, 1.0, 0.0).astype(_f32)
                    kif = jnp.where(ki, 1.0, 0.0).astype(_f32)
                    plsc.addupdate_scatter(deg_v, [ri_g], kuf)
                    plsc.addupdate_scatter(deg_v, [ci_g], kif)

        lin_issue(0, 0)
        lin_issue(1, 1)
        lin_wait(0)
        gat_issue(0, 0)

        def pair(m, _):
            for ph in range(2):
                n = m * 2 + ph
                p = ph
                q = 1 - ph
                gat_wait(p)

                @pl.when(n + 1 < K1_CH)
                def _():
                    lin_wait(q)
                    gat_issue(n + 1, q)

                @pl.when(n >= 2)
                def _():
                    out_wait(p)

                compute(n, p)
                out_issue(n, p)

                @pl.when(n + 2 < K1_CH)
                def _():
                    lin_issue(n + 2, p)
            return 0

        lax.fori_loop(0, K1_CH // 2, pair, 0)
        out_wait(0)
        out_wait(1)
        pltpu.sync_copy(deg_v, degp_h.at[wid])

    return k1(x, p2u, p2i, ipack, l0pack)


def _k3_sc(xs, rsel, colglb):
    mesh = plsc.VectorSubcoreMesh(core_axis_name="c", subcore_axis_name="s")

    @functools.partial(
        pl.kernel,
        out_type=jax.ShapeDtypeStruct((XP, KD), _f32),
        mesh=mesh,
        compiler_params=pltpu.CompilerParams(
            needs_layout_passes=False, use_tc_tiling_on_sc=False),
        scratch_types=[
            pltpu.VMEM((CH, KD), _f32),
            pltpu.VMEM((CH, KD), _f32),
            pltpu.VMEM((2, CH), _i32),
            pltpu.VMEM((2, CH), _i32),
            pltpu.VMEM((2, CH), _i32),
            pltpu.VMEM_SHARED((PAD_HALF, KD), _f32),
            pltpu.SemaphoreType.DMA,
            pltpu.SemaphoreType.DMA,
            pltpu.SemaphoreType.DMA,
            pltpu.SemaphoreType.DMA,
            pltpu.SemaphoreType.DMA,
            pltpu.SemaphoreType.DMA,
        ],
    )
    def k3(xs_h, rsel_h, cglb_h, out_h,
           xc0, xc1, rl_v, cg_v, rs_v, acc_sh,
           sl0, sl1, sg0, sg1, ss0, ss1):
        c = lax.axis_index("c")
        s = lax.axis_index("s")
        obase = jnp.where(c == 0, PAD_HALF, 0).astype(_i32)
        zero16 = jnp.zeros((16,), _f32)
        xc = (xc0, xc1)
        sl = (sl0, sl1)
        sg = (sg0, sg1)
        ss = (ss0, ss1)

        def zb(j, _):
            xc0[j, pl.ds(0, 16)] = zero16
            xc0[j, pl.ds(16, 16)] = zero16
            xc0[j, pl.ds(32, 16)] = zero16
            xc0[j, pl.ds(48, 16)] = zero16
            return 0

        lax.fori_loop(0, CH, zb, 0)
        for i in range(ACC_STRIPE // CH):
            pltpu.sync_copy(xc0, acc_sh.at[pl.ds(s * ACC_STRIPE + i * CH, CH)])
        rem = ACC_STRIPE % CH
        if rem:
            pltpu.sync_copy(
                xc0.at[pl.ds(0, rem)],
                acc_sh.at[pl.ds(s * ACC_STRIPE + (ACC_STRIPE // CH) * CH, rem)])
        plsc.subcore_barrier()

        base = s * K3_CH

        def lin_issue(n, p):
            eoff = (base + n) * CH
            pltpu.async_copy(rsel_h.at[c, pl.ds(eoff, CH)], rl_v.at[p], sl[p])
            pltpu.async_copy(cglb_h.at[c, pl.ds(eoff, CH)], cg_v.at[p], sl[p])

        def lin_wait(p):
            pltpu.make_async_copy(rsel_h.at[0, pl.ds(0, CH)], rl_v.at[p], sl[p]).wait()
            pltpu.make_async_copy(cglb_h.at[0, pl.ds(0, CH)], cg_v.at[p], sl[p]).wait()

        def gat_issue(p):
            pltpu.async_copy(xs_h.at[cg_v.at[p]], xc[p], sg[p])

        def gat_wait(p):
            pltpu.make_async_copy(xs_h.at[cg_v.at[p]], xc[p], sg[p]).wait()

        def sc_issue(p):
            pltpu.async_copy(xc[p], acc_sh.at[rs_v.at[p]], ss[p], add=True)

        def sc_wait(p):
            pltpu.make_async_copy(xc[p], acc_sh.at[rs_v.at[p]], ss[p]).wait()

        lin_issue(0, 0)
        lin_issue(1, 1)
        lin_wait(0)
        gat_issue(0)

        def pair(m, _):
            for ph in range(2):
                n = m * 2 + ph
                p = ph
                q = 1 - ph
                gat_wait(p)
                for g in range(CH // 16):
                    rs_v[p, pl.ds(g * 16, 16)] = rl_v[p, pl.ds(g * 16, 16)]
                sc_issue(p)

                @pl.when(n + 1 < K3_CH)
                def _():
                    lin_wait(q)

                @pl.when(n >= 1)
                def _():
                    sc_wait(q)

                @pl.when(n + 1 < K3_CH)
                def _():
                    gat_issue(q)

                @pl.when(n + 2 < K3_CH)
                def _():
                    lin_issue(n + 2, p)
            return 0

        lax.fori_loop(0, K3_CH // 2, pair, 0)
        sc_wait(1)
        plsc.subcore_barrier()
        pltpu.sync_copy(
            acc_sh.at[pl.ds(s * ACC_STRIPE, ACC_STRIPE)],
            out_h.at[pl.ds(obase + s * ACC_STRIPE, ACC_STRIPE)])

    return k3(xs, rsel, colglb)


def kernel(Gu, Gi, edge_features, Wu, bu, Wi, bi, L0, rows, cols):
    r_item = rows[:EH]
    u_col = cols[:EH]
    item_pg = r_item + (PAD_HALF - NU)
    b_idx = r_item - NU

    def pad1(a, v, dt):
        return jnp.concatenate([a.astype(dt),
                                jnp.full((EP - EH,), v, dt)])

    r1 = pad1(item_pg, PAD_HALF, _i32)
    bix = pad1(b_idx, 0, _i32)
    cu = pad1(u_col, 0, _i32)
    l0u = pad1(L0[:EH], 0.0, _f32)
    l0i = pad1(L0[EH:], 0.0, _f32)

    ipack = jnp.concatenate(
        [r1.reshape(NCHUNK, CH), bix.reshape(NCHUNK, CH),
         cu.reshape(NCHUNK, CH)], axis=1)
    l0pack = jnp.concatenate(
        [l0u.reshape(NCHUNK, CH), l0i.reshape(NCHUNK, CH)], axis=1)
    colglb = jnp.stack([cu, r1])

    x = jnp.zeros((XP, KD), _f32).at[0:NU].set(Gu) \
        .at[PAD_HALF:PAD_HALF + NU].set(Gi)

    p2u, p2i = _p2_tc(edge_features.astype(_f32), Wu, bu, Wi, bi)

    for layer in range(2):
        rsel, degp = _k1_sc(x, p2u, p2i, ipack, l0pack)
        dis2d = _dis_tc(degp)
        xs = _rowscale_tc(x, dis2d)
        raw = _k3_sc(xs, rsel, colglb)
        if layer == 0:
            x = _rowscale_tc(raw, dis2d)

    return _final_tc(raw, dis2d)


# speedup vs baseline: 17.5022x; 1.0063x over previous
import functools

import jax
import jax.numpy as jnp
from jax import lax
from jax.experimental import pallas as pl
from jax.experimental.pallas import tpu as pltpu
from jax.experimental.pallas import tpu_sc as plsc

NU = 25000
PAD_HALF = 25600
XP = 2 * PAD_HALF
KD = 64
EH = 400000
EP = 401408
NCHUNK = EP // 128
CH = 128
K1_CH = EP // (32 * CH)
K1_CPA = 64
K1_CPB = K1_CH - K1_CPA
EA = 32 * K1_CPA * CH
EB = EP - EA
K3_CH = EP // (16 * CH)
ACC_STRIPE = PAD_HALF // 16
DUMP = PAD_HALF - 1

_f32 = jnp.float32
_i32 = jnp.int32


def _p2_tc(ef, Wu, bu, Wi, bi, blk0, nrows):
    nblk = nrows // 2048

    def body(ef_ref, wu_ref, bu_ref, wi_ref, bi_ref, pu_ref, pi_ref):
        e = ef_ref[...]
        pu = jnp.dot(e, wu_ref[...], preferred_element_type=_f32) + bu_ref[...]
        pi = jnp.dot(e, wi_ref[...], preferred_element_type=_f32) + bi_ref[...]
        pu_ref[...] = pu * pu
        pi_ref[...] = pi * pi

    return pl.pallas_call(
        body,
        grid=(nblk,),
        in_specs=[
            pl.BlockSpec((2048, 16), lambda i: (blk0 + i, 0)),
            pl.BlockSpec((16, KD), lambda i: (0, 0)),
            pl.BlockSpec((1, KD), lambda i: (0, 0)),
            pl.BlockSpec((16, KD), lambda i: (0, 0)),
            pl.BlockSpec((1, KD), lambda i: (0, 0)),
        ],
        out_specs=[pl.BlockSpec((2048, KD), lambda i: (i, 0))] * 2,
        out_shape=[jax.ShapeDtypeStruct((nrows, KD), _f32)] * 2,
    )(ef, Wu, bu.reshape(1, KD), Wi, bi.reshape(1, KD))


def _final_tc(raw, dis2d):
    blk = 1600
    nblk = PAD_HALF // blk

    def body(xu_ref, du_ref, xi_ref, di_ref, ou_ref, oi_ref):
        ou_ref[...] = xu_ref[...] * du_ref[...]
        oi_ref[...] = xi_ref[...] * di_ref[...]

    return pl.pallas_call(
        body,
        grid=(nblk,),
        in_specs=[
            pl.BlockSpec((blk, KD), lambda i: (i, 0)),
            pl.BlockSpec((blk, 1), lambda i: (i, 0)),
            pl.BlockSpec((blk, KD), lambda i: (nblk + i, 0)),
            pl.BlockSpec((blk, 1), lambda i: (nblk + i, 0)),
        ],
        out_specs=[pl.BlockSpec((blk, KD), lambda i: (i, 0))] * 2,
        out_shape=[jax.ShapeDtypeStruct((NU, KD), _f32)] * 2,
    )(raw, dis2d, raw, dis2d)


def _dis_tc(degp_a, degp_b):
    def body(da_ref, db_ref, o_ref):
        d = jnp.sum(da_ref[...], axis=0) + jnp.sum(db_ref[...], axis=0)
        o_ref[...] = jnp.where(d > 0, 1.0 / jnp.sqrt(d), 0.0)

    out = pl.pallas_call(
        body,
        out_shape=jax.ShapeDtypeStruct((XP // 128, 128), _f32),
    )(degp_a.reshape(32, XP // 128, 128), degp_b.reshape(32, XP // 128, 128))
    return out.reshape(XP, 1)


def _rowscale_tc(x, dis2d):
    nblk = XP // 2048

    def body(x_ref, d_ref, o_ref):
        o_ref[...] = x_ref[...] * d_ref[...]

    return pl.pallas_call(
        body,
        grid=(nblk,),
        in_specs=[
            pl.BlockSpec((2048, KD), lambda i: (i, 0)),
            pl.BlockSpec((2048, 1), lambda i: (i, 0)),
        ],
        out_specs=pl.BlockSpec((2048, KD), lambda i: (i, 0)),
        out_shape=jax.ShapeDtypeStruct((XP, KD), _f32),
    )(x, dis2d)


def _k1_sc(x, p2u, p2i, ipack, l0pack, chunk0, cpt):
    epart = 32 * cpt * CH
    mesh = plsc.VectorSubcoreMesh(core_axis_name="c", subcore_axis_name="s")

    @functools.partial(
        pl.kernel,
        out_type=[
            jax.ShapeDtypeStruct((2, epart), _i32),
            jax.ShapeDtypeStruct((32, XP), _f32),
        ],
        mesh=mesh,
        compiler_params=pltpu.CompilerParams(
            needs_layout_passes=False, use_tc_tiling_on_sc=False),
        scratch_types=[
            pltpu.VMEM((2, 384), _i32),
            pltpu.VMEM((2, 256), _f32),
            pltpu.VMEM((CH, KD), _f32),
            pltpu.VMEM((CH, KD), _f32),
            pltpu.VMEM((CH, KD), _f32),
            pltpu.VMEM((CH, KD), _f32),
            pltpu.VMEM((CH, KD), _f32),
            pltpu.VMEM((CH, KD), _f32),
            pltpu.VMEM((CH, KD), _f32),
            pltpu.VMEM((CH, KD), _f32),
            pltpu.VMEM((2, CH), _i32),
            pltpu.VMEM((2, CH), _i32),
            pltpu.VMEM((XP,), _f32),
            pltpu.SemaphoreType.DMA,
            pltpu.SemaphoreType.DMA,
            pltpu.SemaphoreType.DMA,
            pltpu.SemaphoreType.DMA,
            pltpu.SemaphoreType.DMA,
            pltpu.SemaphoreType.DMA,
        ],
    )
    def k1(x_h, pu_h, pi_h, ipk_h, l0_h,
           rsel_h, degp_h,
           ipk_v, l0_v, a0, a1, b0, b1, u0, u1, i0, i1,
           ru_v, ri_v, deg_v,
           sl0, sl1, sg0, sg1, so0, so1):
        c = lax.axis_index("c")
        s = lax.axis_index("s")
        wid = c * 16 + s
        iota = lax.iota(_i32, 16)
        zero16 = jnp.zeros((16,), _f32)
        a_v = (a0, a1)
        b_v = (b0, b1)
        pu_v = (u0, u1)
        pi_v = (i0, i1)
        sl = (sl0, sl1)
        sg = (sg0, sg1)
        so = (so0, so1)

        def zdeg(i, _):
            deg_v[pl.ds(i * 16, 16)] = zero16
            return 0

        lax.fori_loop(0, XP // 16, zdeg, 0)

        lbase = wid * cpt
        base = chunk0 + lbase

        def lin_issue(n, p):
            pltpu.async_copy(ipk_h.at[base + n], ipk_v.at[p], sl[p])
            pltpu.async_copy(l0_h.at[base + n], l0_v.at[p], sl[p])

        def lin_wait(p):
            pltpu.make_async_copy(ipk_h.at[0], ipk_v.at[p], sl[p]).wait()
            pltpu.make_async_copy(l0_h.at[0], l0_v.at[p], sl[p]).wait()

        def gat_issue(n, p):
            eoff = (lbase + n) * CH
            pltpu.async_copy(x_h.at[ipk_v.at[p, pl.ds(0, CH)]], a_v[p], sg[p])
            pltpu.async_copy(x_h.at[ipk_v.at[p, pl.ds(CH, CH)]], b_v[p], sg[p])
            pltpu.async_copy(pu_h.at[pl.ds(eoff, CH)], pu_v[p], sg[p])
            pltpu.async_copy(pi_h.at[pl.ds(eoff, CH)], pi_v[p], sg[p])

        def gat_wait(p):
            pltpu.make_async_copy(x_h.at[ipk_v.at[p, pl.ds(0, CH)]], a_v[p], sg[p]).wait()
            pltpu.make_async_copy(x_h.at[ipk_v.at[p, pl.ds(CH, CH)]], b_v[p], sg[p]).wait()
            pltpu.make_async_copy(pu_h.at[pl.ds(0, CH)], pu_v[p], sg[p]).wait()
            pltpu.make_async_copy(pi_h.at[pl.ds(0, CH)], pi_v[p], sg[p]).wait()

        def out_issue(n, p):
            eoff = (lbase + n) * CH
            pltpu.async_copy(ru_v.at[p], rsel_h.at[0, pl.ds(eoff, CH)], so[p])
            pltpu.async_copy(ri_v.at[p], rsel_h.at[1, pl.ds(eoff, CH)], so[p])

        def out_wait(p):
            pltpu.make_async_copy(ru_v.at[p], rsel_h.at[0, pl.ds(0, CH)], so[p]).wait()
            pltpu.make_async_copy(ri_v.at[p], rsel_h.at[1, pl.ds(0, CH)], so[p]).wait()

        def compute(n, p):
            eoff = (base + n) * CH
            for half in range(2):
                gset = [half * 4 + gg for gg in range(4)]
                rowis = [g * 16 + iota for g in gset]

                def dot_k(k, acc):
                    acc = list(acc)
                    ck = (iota + k) & (KD - 1)
                    va = [plsc.load_gather(a_v[p], [rowis[gg], ck])
                          for gg in range(4)]
                    vb = [plsc.load_gather(b_v[p], [rowis[gg], ck])
                          for gg in range(4)]
                    vu = [plsc.load_gather(pu_v[p], [rowis[gg], ck])
                          for gg in range(4)]
                    vi = [plsc.load_gather(pi_v[p], [rowis[gg], ck])
                          for gg in range(4)]
                    for gg in range(4):
                        ab = va[gg] * vb[gg]
                        aa = va[gg] * va[gg]
                        bb = vb[gg] * vb[gg]
                        o = gg * 6
                        acc[o + 0] = acc[o + 0] + ab * vu[gg]
                        acc[o + 1] = acc[o + 1] + aa * vu[gg]
                        acc[o + 2] = acc[o + 2] + bb * vu[gg]
                        acc[o + 3] = acc[o + 3] + ab * vi[gg]
                        acc[o + 4] = acc[o + 4] + aa * vi[gg]
                        acc[o + 5] = acc[o + 5] + bb * vi[gg]
                    return tuple(acc)

                z = jnp.zeros((16,), _f32)
                accs = lax.fori_loop(0, KD, dot_k, (z,) * 24)
                for gg in range(4):
                    g = gset[gg]
                    abu, aau, bbu, abi, aai, bbi = accs[gg * 6:gg * 6 + 6]
                    tl_u = 0.2 - l0_v[p, pl.ds(g * 16, 16)]
                    tl_i = 0.2 - l0_v[p, pl.ds(CH + g * 16, 16)]
                    e2 = jnp.float32(1e-16)
                    ku = ((tl_u <= 0)
                          | ((abu > 0)
                             & (abu * abu >= tl_u * tl_u * jnp.maximum(aau, e2)
                                * jnp.maximum(bbu, e2))))
                    ki = ((tl_i <= 0)
                          | ((abi > 0)
                             & (abi * abi >= tl_i * tl_i * jnp.maximum(aai, e2)
                                * jnp.maximum(bbi, e2))))
                    valid = (eoff + g * 16 + iota) < EH
                    ku = ku & valid
                    ki = ki & valid
                    bi_g = ipk_v[p, pl.ds(CH + g * 16, 16)]
                    ci_g = ipk_v[p, pl.ds(2 * CH + g * 16, 16)]
                    ri_g = ipk_v[p, pl.ds(g * 16, 16)]
                    ru_v[p, pl.ds(g * 16, 16)] = jnp.where(ku, bi_g, DUMP)
                    ri_v[p, pl.ds(g * 16, 16)] = jnp.where(ki, ci_g, DUMP)
                    kuf = jnp.where(ku, 1.0, 0.0).astype(_f32)
                    kif = jnp.where(ki, 1.0, 0.0).astype(_f32)
                    plsc.addupdate_scatter(deg_v, [ri_g], kuf)
                    plsc.addupdate_scatter(deg_v, [ci_g], kif)

        lin_issue(0, 0)
        lin_issue(1, 1)
        lin_wait(0)
        gat_issue(0, 0)

        def pair(m, _):
            for ph in range(2):
                n = m * 2 + ph
                p = ph
                q = 1 - ph
                gat_wait(p)

                @pl.when(n + 1 < cpt)
                def _():
                    lin_wait(q)
                    gat_issue(n + 1, q)

                @pl.when(n >= 2)
                def _():
                    out_wait(p)

                compute(n, p)
                out_issue(n, p)

                @pl.when(n + 2 < cpt)
                def _():
                    lin_issue(n + 2, p)
            return 0

        lax.fori_loop(0, cpt // 2, pair, 0)
        out_wait(0)
        out_wait(1)
        pltpu.sync_copy(deg_v, degp_h.at[wid])

    return k1(x, p2u, p2i, ipack, l0pack)


def _k3_sc(xs, rsel, colglb):
    mesh = plsc.VectorSubcoreMesh(core_axis_name="c", subcore_axis_name="s")

    @functools.partial(
        pl.kernel,
        out_type=jax.ShapeDtypeStruct((XP, KD), _f32),
        mesh=mesh,
        compiler_params=pltpu.CompilerParams(
            needs_layout_passes=False, use_tc_tiling_on_sc=False),
        scratch_types=[
            pltpu.VMEM((CH, KD), _f32),
            pltpu.VMEM((CH, KD), _f32),
            pltpu.VMEM((2, CH), _i32),
            pltpu.VMEM((2, CH), _i32),
            pltpu.VMEM((2, CH), _i32),
            pltpu.VMEM_SHARED((PAD_HALF, KD), _f32),
            pltpu.SemaphoreType.DMA,
            pltpu.SemaphoreType.DMA,
            pltpu.SemaphoreType.DMA,
            pltpu.SemaphoreType.DMA,
            pltpu.SemaphoreType.DMA,
            pltpu.SemaphoreType.DMA,
        ],
    )
    def k3(xs_h, rsel_h, cglb_h, out_h,
           xc0, xc1, rl_v, cg_v, rs_v, acc_sh,
           sl0, sl1, sg0, sg1, ss0, ss1):
        c = lax.axis_index("c")
        s = lax.axis_index("s")
        obase = jnp.where(c == 0, PAD_HALF, 0).astype(_i32)
        zero16 = jnp.zeros((16,), _f32)
        xc = (xc0, xc1)
        sl = (sl0, sl1)
        sg = (sg0, sg1)
        ss = (ss0, ss1)

        def zb(j, _):
            xc0[j, pl.ds(0, 16)] = zero16
            xc0[j, pl.ds(16, 16)] = zero16
            xc0[j, pl.ds(32, 16)] = zero16
            xc0[j, pl.ds(48, 16)] = zero16
            return 0

        lax.fori_loop(0, CH, zb, 0)
        for i in range(ACC_STRIPE // CH):
            pltpu.sync_copy(xc0, acc_sh.at[pl.ds(s * ACC_STRIPE + i * CH, CH)])
        rem = ACC_STRIPE % CH
        if rem:
            pltpu.sync_copy(
                xc0.at[pl.ds(0, rem)],
                acc_sh.at[pl.ds(s * ACC_STRIPE + (ACC_STRIPE // CH) * CH, rem)])
        plsc.subcore_barrier()

        base = s * K3_CH

        def lin_issue(n, p):
            eoff = (base + n) * CH
            pltpu.async_copy(rsel_h.at[c, pl.ds(eoff, CH)], rl_v.at[p], sl[p])
            pltpu.async_copy(cglb_h.at[c, pl.ds(eoff, CH)], cg_v.at[p], sl[p])

        def lin_wait(p):
            pltpu.make_async_copy(rsel_h.at[0, pl.ds(0, CH)], rl_v.at[p], sl[p]).wait()
            pltpu.make_async_copy(cglb_h.at[0, pl.ds(0, CH)], cg_v.at[p], sl[p]).wait()

        def gat_issue(p):
            pltpu.async_copy(xs_h.at[cg_v.at[p]], xc[p], sg[p])

        def gat_wait(p):
            pltpu.make_async_copy(xs_h.at[cg_v.at[p]], xc[p], sg[p]).wait()

        def sc_issue(p):
            pltpu.async_copy(xc[p], acc_sh.at[rs_v.at[p]], ss[p], add=True)

        def sc_wait(p):
            pltpu.make_async_copy(xc[p], acc_sh.at[rs_v.at[p]], ss[p]).wait()

        lin_issue(0, 0)
        lin_issue(1, 1)
        lin_wait(0)
        gat_issue(0)

        def pair(m, _):
            for ph in range(2):
                n = m * 2 + ph
                p = ph
                q = 1 - ph
                gat_wait(p)
                for g in range(CH // 16):
                    rs_v[p, pl.ds(g * 16, 16)] = rl_v[p, pl.ds(g * 16, 16)]
                sc_issue(p)

                @pl.when(n + 1 < K3_CH)
                def _():
                    lin_wait(q)

                @pl.when(n >= 1)
                def _():
                    sc_wait(q)

                @pl.when(n + 1 < K3_CH)
                def _():
                    gat_issue(q)

                @pl.when(n + 2 < K3_CH)
                def _():
                    lin_issue(n + 2, p)
            return 0

        lax.fori_loop(0, K3_CH // 2, pair, 0)
        sc_wait(1)
        plsc.subcore_barrier()
        pltpu.sync_copy(
            acc_sh.at[pl.ds(s * ACC_STRIPE, ACC_STRIPE)],
            out_h.at[pl.ds(obase + s * ACC_STRIPE, ACC_STRIPE)])

    return k3(xs, rsel, colglb)


def kernel(Gu, Gi, edge_features, Wu, bu, Wi, bi, L0, rows, cols):
    r_item = rows[:EH]
    u_col = cols[:EH]
    item_pg = r_item + (PAD_HALF - NU)
    b_idx = r_item - NU

    def pad1(a, v, dt):
        return jnp.concatenate([a.astype(dt),
                                jnp.full((EP - EH,), v, dt)])

    r1 = pad1(item_pg, PAD_HALF, _i32)
    bix = pad1(b_idx, 0, _i32)
    cu = pad1(u_col, 0, _i32)
    l0u = pad1(L0[:EH], 0.0, _f32)
    l0i = pad1(L0[EH:], 0.0, _f32)

    ipack = jnp.concatenate(
        [r1.reshape(NCHUNK, CH), bix.reshape(NCHUNK, CH),
         cu.reshape(NCHUNK, CH)], axis=1)
    l0pack = jnp.concatenate(
        [l0u.reshape(NCHUNK, CH), l0i.reshape(NCHUNK, CH)], axis=1)
    colglb = jnp.stack([cu, r1])

    x = jnp.zeros((XP, KD), _f32).at[0:NU].set(Gu) \
        .at[PAD_HALF:PAD_HALF + NU].set(Gi)

    ef = edge_features.astype(_f32)
    p2u_a, p2i_a = _p2_tc(ef, Wu, bu, Wi, bi, 0, EA)
    p2u_b, p2i_b = _p2_tc(ef, Wu, bu, Wi, bi, EA // 2048, EB)

    for layer in range(2):
        rsel_a, degp_a = _k1_sc(x, p2u_a, p2i_a, ipack, l0pack, 0, K1_CPA)
        rsel_b, degp_b = _k1_sc(x, p2u_b, p2i_b, ipack, l0pack,
                                EA // CH, K1_CPB)
        rsel = jnp.concatenate([rsel_a, rsel_b], axis=1)
        dis2d = _dis_tc(degp_a, degp_b)
        xs = _rowscale_tc(x, dis2d)
        raw = _k3_sc(xs, rsel, colglb)
        if layer == 0:
            x = _rowscale_tc(raw, dis2d)

    return _final_tc(raw, dis2d)


# speedup vs baseline: 18.0181x; 1.0295x over previous
import functools

import jax
import jax.numpy as jnp
from jax import lax
from jax.experimental import pallas as pl
from jax.experimental.pallas import tpu as pltpu
from jax.experimental.pallas import tpu_sc as plsc

NU = 25000
PAD_HALF = 25600
XP = 2 * PAD_HALF
KD = 64
EH = 400000
EP = 401408
NCHUNK = EP // 128
CH = 128
K1_CH = EP // (32 * CH)
K1_CPA = 64
K1_CPB = K1_CH - K1_CPA
EA = 32 * K1_CPA * CH
EB = EP - EA
K3_CH = EP // (16 * CH)
ACC_STRIPE = PAD_HALF // 16
DUMP = PAD_HALF - 1

_f32 = jnp.float32
_i32 = jnp.int32


def _p2_tc(ef, Wu, bu, Wi, bi, blk0, nrows):
    nblk = nrows // 2048

    def body(ef_ref, wu_ref, bu_ref, wi_ref, bi_ref, pu_ref, pi_ref):
        e = ef_ref[...]
        pu = jnp.dot(e, wu_ref[...], preferred_element_type=_f32) + bu_ref[...]
        pi = jnp.dot(e, wi_ref[...], preferred_element_type=_f32) + bi_ref[...]
        pu_ref[...] = pu * pu
        pi_ref[...] = pi * pi

    return pl.pallas_call(
        body,
        grid=(nblk,),
        in_specs=[
            pl.BlockSpec((2048, 16), lambda i: (blk0 + i, 0)),
            pl.BlockSpec((16, KD), lambda i: (0, 0)),
            pl.BlockSpec((1, KD), lambda i: (0, 0)),
            pl.BlockSpec((16, KD), lambda i: (0, 0)),
            pl.BlockSpec((1, KD), lambda i: (0, 0)),
        ],
        out_specs=[pl.BlockSpec((2048, KD), lambda i: (i, 0))] * 2,
        out_shape=[jax.ShapeDtypeStruct((nrows, KD), _f32)] * 2,
    )(ef, Wu, bu.reshape(1, KD), Wi, bi.reshape(1, KD))


def _final_tc(raw, dis2d):
    blk = 1600
    nblk = PAD_HALF // blk

    def body(xu_ref, du_ref, xi_ref, di_ref, ou_ref, oi_ref):
        ou_ref[...] = xu_ref[...] * du_ref[...]
        oi_ref[...] = xi_ref[...] * di_ref[...]

    return pl.pallas_call(
        body,
        grid=(nblk,),
        in_specs=[
            pl.BlockSpec((blk, KD), lambda i: (i, 0)),
            pl.BlockSpec((blk, 1), lambda i: (i, 0)),
            pl.BlockSpec((blk, KD), lambda i: (nblk + i, 0)),
            pl.BlockSpec((blk, 1), lambda i: (nblk + i, 0)),
        ],
        out_specs=[pl.BlockSpec((blk, KD), lambda i: (i, 0))] * 2,
        out_shape=[jax.ShapeDtypeStruct((NU, KD), _f32)] * 2,
    )(raw, dis2d, raw, dis2d)


def _dis_tc(*degps):
    def body(*refs):
        o_ref = refs[-1]
        d = refs[0][...].sum(axis=0)
        for r in refs[1:-1]:
            d = d + r[...].sum(axis=0)
        o_ref[...] = jnp.where(d > 0, 1.0 / jnp.sqrt(d), 0.0)

    out = pl.pallas_call(
        body,
        out_shape=jax.ShapeDtypeStruct((XP // 128, 128), _f32),
    )(*[dp.reshape(32, XP // 128, 128) for dp in degps])
    return out.reshape(XP, 1)


def _rowscale_tc(x, dis2d):
    nblk = XP // 2048

    def body(x_ref, d_ref, o_ref):
        o_ref[...] = x_ref[...] * d_ref[...]

    return pl.pallas_call(
        body,
        grid=(nblk,),
        in_specs=[
            pl.BlockSpec((2048, KD), lambda i: (i, 0)),
            pl.BlockSpec((2048, 1), lambda i: (i, 0)),
        ],
        out_specs=pl.BlockSpec((2048, KD), lambda i: (i, 0)),
        out_shape=jax.ShapeDtypeStruct((XP, KD), _f32),
    )(x, dis2d)


def _k1_sc(x, p2u, p2i, ipack, l0pack, chunk0, cpt, p2u2=None, p2i2=None):
    epart = 32 * cpt * CH
    dual = p2u2 is not None
    extra = (p2u2, p2i2) if dual else ()
    mesh = plsc.VectorSubcoreMesh(core_axis_name="c", subcore_axis_name="s")

    @functools.partial(
        pl.kernel,
        out_type=[
            jax.ShapeDtypeStruct((2, epart), _i32),
            jax.ShapeDtypeStruct((32, XP), _f32),
        ],
        mesh=mesh,
        compiler_params=pltpu.CompilerParams(
            needs_layout_passes=False, use_tc_tiling_on_sc=False),
        scratch_types=[
            pltpu.VMEM((2, 384), _i32),
            pltpu.VMEM((2, 256), _f32),
            pltpu.VMEM((CH, KD), _f32),
            pltpu.VMEM((CH, KD), _f32),
            pltpu.VMEM((CH, KD), _f32),
            pltpu.VMEM((CH, KD), _f32),
            pltpu.VMEM((CH, KD), _f32),
            pltpu.VMEM((CH, KD), _f32),
            pltpu.VMEM((CH, KD), _f32),
            pltpu.VMEM((CH, KD), _f32),
            pltpu.VMEM((2, CH), _i32),
            pltpu.VMEM((2, CH), _i32),
            pltpu.VMEM((XP,), _f32),
            pltpu.SemaphoreType.DMA,
            pltpu.SemaphoreType.DMA,
            pltpu.SemaphoreType.DMA,
            pltpu.SemaphoreType.DMA,
            pltpu.SemaphoreType.DMA,
            pltpu.SemaphoreType.DMA,
        ],
    )
    def k1(x_h, pu_h, pi_h, pu2_h, pi2_h, ipk_h, l0_h,
           rsel_h, degp_h,
           ipk_v, l0_v, a0, a1, b0, b1, u0, u1, i0, i1,
           ru_v, ri_v, deg_v,
           sl0, sl1, sg0, sg1, so0, so1):
        c = lax.axis_index("c")
        s = lax.axis_index("s")
        wid = c * 16 + s
        iota = lax.iota(_i32, 16)
        zero16 = jnp.zeros((16,), _f32)
        a_v = (a0, a1)
        b_v = (b0, b1)
        pu_v = (u0, u1)
        pi_v = (i0, i1)
        sl = (sl0, sl1)
        sg = (sg0, sg1)
        so = (so0, so1)

        def zdeg(i, _):
            for u in range(8):
                deg_v[pl.ds(i * 128 + u * 16, 16)] = zero16
            return 0

        lax.fori_loop(0, XP // 128, zdeg, 0)

        lbase = wid * cpt
        base = chunk0 + lbase

        def lin_issue(n, p):
            pltpu.async_copy(ipk_h.at[base + n], ipk_v.at[p], sl[p])
            pltpu.async_copy(l0_h.at[base + n], l0_v.at[p], sl[p])

        def lin_wait(p):
            pltpu.make_async_copy(ipk_h.at[0], ipk_v.at[p], sl[p]).wait()
            pltpu.make_async_copy(l0_h.at[0], l0_v.at[p], sl[p]).wait()

        def gat_issue(n, p):
            pltpu.async_copy(x_h.at[ipk_v.at[p, pl.ds(0, CH)]], a_v[p], sg[p])
            pltpu.async_copy(x_h.at[ipk_v.at[p, pl.ds(CH, CH)]], b_v[p], sg[p])
            if dual:
                g = base + n

                @pl.when(g < EA // CH)
                def _():
                    eoff = g * CH
                    pltpu.async_copy(pu_h.at[pl.ds(eoff, CH)], pu_v[p], sg[p])
                    pltpu.async_copy(pi_h.at[pl.ds(eoff, CH)], pi_v[p], sg[p])

                @pl.when(g >= EA // CH)
                def _():
                    eoff = (g - EA // CH) * CH
                    pltpu.async_copy(pu2_h.at[pl.ds(eoff, CH)], pu_v[p], sg[p])
                    pltpu.async_copy(pi2_h.at[pl.ds(eoff, CH)], pi_v[p], sg[p])
            else:
                eoff = (lbase + n) * CH
                pltpu.async_copy(pu_h.at[pl.ds(eoff, CH)], pu_v[p], sg[p])
                pltpu.async_copy(pi_h.at[pl.ds(eoff, CH)], pi_v[p], sg[p])

        def gat_wait(p):
            pltpu.make_async_copy(x_h.at[ipk_v.at[p, pl.ds(0, CH)]], a_v[p], sg[p]).wait()
            pltpu.make_async_copy(x_h.at[ipk_v.at[p, pl.ds(CH, CH)]], b_v[p], sg[p]).wait()
            pltpu.make_async_copy(pu_h.at[pl.ds(0, CH)], pu_v[p], sg[p]).wait()
            pltpu.make_async_copy(pi_h.at[pl.ds(0, CH)], pi_v[p], sg[p]).wait()

        def out_issue(n, p):
            eoff = (lbase + n) * CH
            pltpu.async_copy(ru_v.at[p], rsel_h.at[0, pl.ds(eoff, CH)], so[p])
            pltpu.async_copy(ri_v.at[p], rsel_h.at[1, pl.ds(eoff, CH)], so[p])

        def out_wait(p):
            pltpu.make_async_copy(ru_v.at[p], rsel_h.at[0, pl.ds(0, CH)], so[p]).wait()
            pltpu.make_async_copy(ri_v.at[p], rsel_h.at[1, pl.ds(0, CH)], so[p]).wait()

        def compute(n, p):
            eoff = (base + n) * CH
            for half in range(2):
                gset = [half * 4 + gg for gg in range(4)]
                rowis = [g * 16 + iota for g in gset]

                def dot_k(k, acc):
                    acc = list(acc)
                    ck = (iota + k) & (KD - 1)
                    va = [plsc.load_gather(a_v[p], [rowis[gg], ck])
                          for gg in range(4)]
                    vb = [plsc.load_gather(b_v[p], [rowis[gg], ck])
                          for gg in range(4)]
                    vu = [plsc.load_gather(pu_v[p], [rowis[gg], ck])
                          for gg in range(4)]
                    vi = [plsc.load_gather(pi_v[p], [rowis[gg], ck])
                          for gg in range(4)]
                    for gg in range(4):
                        ab = va[gg] * vb[gg]
                        aa = va[gg] * va[gg]
                        bb = vb[gg] * vb[gg]
                        o = gg * 6
                        acc[o + 0] = acc[o + 0] + ab * vu[gg]
                        acc[o + 1] = acc[o + 1] + aa * vu[gg]
                        acc[o + 2] = acc[o + 2] + bb * vu[gg]
                        acc[o + 3] = acc[o + 3] + ab * vi[gg]
                        acc[o + 4] = acc[o + 4] + aa * vi[gg]
                        acc[o + 5] = acc[o + 5] + bb * vi[gg]
                    return tuple(acc)

                z = jnp.zeros((16,), _f32)
                accs = lax.fori_loop(0, KD, dot_k, (z,) * 24)
                for gg in range(4):
                    g = gset[gg]
                    abu, aau, bbu, abi, aai, bbi = accs[gg * 6:gg * 6 + 6]
                    tl_u = 0.2 - l0_v[p, pl.ds(g * 16, 16)]
                    tl_i = 0.2 - l0_v[p, pl.ds(CH + g * 16, 16)]
                    e2 = jnp.float32(1e-16)
                    ku = ((tl_u <= 0)
                          | ((abu > 0)
                             & (abu * abu >= tl_u * tl_u * jnp.maximum(aau, e2)
                                * jnp.maximum(bbu, e2))))
                    ki = ((tl_i <= 0)
                          | ((abi > 0)
                             & (abi * abi >= tl_i * tl_i * jnp.maximum(aai, e2)
                                * jnp.maximum(bbi, e2))))
                    valid = (eoff + g * 16 + iota) < EH
                    ku = ku & valid
                    ki = ki & valid
                    bi_g = ipk_v[p, pl.ds(CH + g * 16, 16)]
                    ci_g = ipk_v[p, pl.ds(2 * CH + g * 16, 16)]
                    ri_g = ipk_v[p, pl.ds(g * 16, 16)]
                    ru_v[p, pl.ds(g * 16, 16)] = jnp.where(ku, bi_g, DUMP)
                    ri_v[p, pl.ds(g * 16, 16)] = jnp.where(ki, ci_g, DUMP)
                    kuf = jnp.where(ku, 1.0, 0.0).astype(_f32)
                    kif = jnp.where(ki, 1.0, 0.0).astype(_f32)
                    plsc.addupdate_scatter(deg_v, [ri_g], kuf)
                    plsc.addupdate_scatter(deg_v, [ci_g], kif)

        lin_issue(0, 0)
        lin_issue(1, 1)
        lin_wait(0)
        gat_issue(0, 0)

        def pair(m, _):
            for ph in range(2):
                n = m * 2 + ph
                p = ph
                q = 1 - ph
                gat_wait(p)

                @pl.when(n + 1 < cpt)
                def _():
                    lin_wait(q)
                    gat_issue(n + 1, q)

                @pl.when(n >= 2)
                def _():
                    out_wait(p)

                compute(n, p)
                out_issue(n, p)

                @pl.when(n + 2 < cpt)
                def _():
                    lin_issue(n + 2, p)
            return 0

        lax.fori_loop(0, cpt // 2, pair, 0)
        out_wait(0)
        out_wait(1)
        pltpu.sync_copy(deg_v, degp_h.at[wid])

    if dual:
        return k1(x, p2u, p2i, p2u2, p2i2, ipack, l0pack)
    return k1(x, p2u, p2i, p2u, p2i, ipack, l0pack)


def _k3_sc(xs, rsel, colglb):
    mesh = plsc.VectorSubcoreMesh(core_axis_name="c", subcore_axis_name="s")

    @functools.partial(
        pl.kernel,
        out_type=jax.ShapeDtypeStruct((XP, KD), _f32),
        mesh=mesh,
        compiler_params=pltpu.CompilerParams(
            needs_layout_passes=False, use_tc_tiling_on_sc=False),
        scratch_types=[
            pltpu.VMEM((CH, KD), _f32),
            pltpu.VMEM((CH, KD), _f32),
            pltpu.VMEM((2, CH), _i32),
            pltpu.VMEM((2, CH), _i32),
            pltpu.VMEM((2, CH), _i32),
            pltpu.VMEM_SHARED((PAD_HALF, KD), _f32),
            pltpu.SemaphoreType.DMA,
            pltpu.SemaphoreType.DMA,
            pltpu.SemaphoreType.DMA,
            pltpu.SemaphoreType.DMA,
            pltpu.SemaphoreType.DMA,
            pltpu.SemaphoreType.DMA,
        ],
    )
    def k3(xs_h, rsel_h, cglb_h, out_h,
           xc0, xc1, rl_v, cg_v, rs_v, acc_sh,
           sl0, sl1, sg0, sg1, ss0, ss1):
        c = lax.axis_index("c")
        s = lax.axis_index("s")
        obase = jnp.where(c == 0, PAD_HALF, 0).astype(_i32)
        zero16 = jnp.zeros((16,), _f32)
        xc = (xc0, xc1)
        sl = (sl0, sl1)
        sg = (sg0, sg1)
        ss = (ss0, ss1)

        def zb(j, _):
            xc0[j, pl.ds(0, 16)] = zero16
            xc0[j, pl.ds(16, 16)] = zero16
            xc0[j, pl.ds(32, 16)] = zero16
            xc0[j, pl.ds(48, 16)] = zero16
            return 0

        lax.fori_loop(0, CH, zb, 0)
        for i in range(ACC_STRIPE // CH):
            pltpu.sync_copy(xc0, acc_sh.at[pl.ds(s * ACC_STRIPE + i * CH, CH)])
        rem = ACC_STRIPE % CH
        if rem:
            pltpu.sync_copy(
                xc0.at[pl.ds(0, rem)],
                acc_sh.at[pl.ds(s * ACC_STRIPE + (ACC_STRIPE // CH) * CH, rem)])
        plsc.subcore_barrier()

        base = s * K3_CH

        def lin_issue(n, p):
            eoff = (base + n) * CH
            pltpu.async_copy(rsel_h.at[c, pl.ds(eoff, CH)], rl_v.at[p], sl[p])
            pltpu.async_copy(cglb_h.at[c, pl.ds(eoff, CH)], cg_v.at[p], sl[p])

        def lin_wait(p):
            pltpu.make_async_copy(rsel_h.at[0, pl.ds(0, CH)], rl_v.at[p], sl[p]).wait()
            pltpu.make_async_copy(cglb_h.at[0, pl.ds(0, CH)], cg_v.at[p], sl[p]).wait()

        def gat_issue(p):
            pltpu.async_copy(xs_h.at[cg_v.at[p]], xc[p], sg[p])

        def gat_wait(p):
            pltpu.make_async_copy(xs_h.at[cg_v.at[p]], xc[p], sg[p]).wait()

        def sc_issue(p):
            pltpu.async_copy(xc[p], acc_sh.at[rs_v.at[p]], ss[p], add=True)

        def sc_wait(p):
            pltpu.make_async_copy(xc[p], acc_sh.at[rs_v.at[p]], ss[p]).wait()

        lin_issue(0, 0)
        lin_issue(1, 1)
        lin_wait(0)
        gat_issue(0)

        def pair(m, _):
            for ph in range(2):
                n = m * 2 + ph
                p = ph
                q = 1 - ph
                gat_wait(p)
                for g in range(CH // 16):
                    rs_v[p, pl.ds(g * 16, 16)] = rl_v[p, pl.ds(g * 16, 16)]
                sc_issue(p)

                @pl.when(n + 1 < K3_CH)
                def _():
                    lin_wait(q)

                @pl.when(n >= 1)
                def _():
                    sc_wait(q)

                @pl.when(n + 1 < K3_CH)
                def _():
                    gat_issue(q)

                @pl.when(n + 2 < K3_CH)
                def _():
                    lin_issue(n + 2, p)
            return 0

        lax.fori_loop(0, K3_CH // 2, pair, 0)
        sc_wait(1)
        plsc.subcore_barrier()
        pltpu.sync_copy(
            acc_sh.at[pl.ds(s * ACC_STRIPE, ACC_STRIPE)],
            out_h.at[pl.ds(obase + s * ACC_STRIPE, ACC_STRIPE)])

    return k3(xs, rsel, colglb)


def kernel(Gu, Gi, edge_features, Wu, bu, Wi, bi, L0, rows, cols):
    r_item = rows[:EH]
    u_col = cols[:EH]
    item_pg = r_item + (PAD_HALF - NU)
    b_idx = r_item - NU

    def pad1(a, v, dt):
        return jnp.concatenate([a.astype(dt),
                                jnp.full((EP - EH,), v, dt)])

    r1 = pad1(item_pg, PAD_HALF, _i32)
    bix = pad1(b_idx, 0, _i32)
    cu = pad1(u_col, 0, _i32)
    l0u = pad1(L0[:EH], 0.0, _f32)
    l0i = pad1(L0[EH:], 0.0, _f32)

    ipack = jnp.concatenate(
        [r1.reshape(NCHUNK, CH), bix.reshape(NCHUNK, CH),
         cu.reshape(NCHUNK, CH)], axis=1)
    l0pack = jnp.concatenate(
        [l0u.reshape(NCHUNK, CH), l0i.reshape(NCHUNK, CH)], axis=1)
    colglb = jnp.stack([cu, r1])

    x = jnp.zeros((XP, KD), _f32).at[0:NU].set(Gu) \
        .at[PAD_HALF:PAD_HALF + NU].set(Gi)

    ef = edge_features.astype(_f32)
    p2u_a, p2i_a = _p2_tc(ef, Wu, bu, Wi, bi, 0, EA)
    p2u_b, p2i_b = _p2_tc(ef, Wu, bu, Wi, bi, EA // 2048, EB)

    for layer in range(2):
        if layer == 0:
            rsel_a, degp_a = _k1_sc(x, p2u_a, p2i_a, ipack, l0pack,
                                    0, K1_CPA)
            rsel_b, degp_b = _k1_sc(x, p2u_b, p2i_b, ipack, l0pack,
                                    EA // CH, K1_CPB)
            rsel = jnp.concatenate([rsel_a, rsel_b], axis=1)
            dis2d = _dis_tc(degp_a, degp_b)
        else:
            rsel, degp = _k1_sc(x, p2u_a, p2i_a, ipack, l0pack,
                                0, K1_CH, p2u_b, p2i_b)
            dis2d = _dis_tc(degp)
        xs = _rowscale_tc(x, dis2d)
        raw = _k3_sc(xs, rsel, colglb)
        if layer == 0:
            x = _rowscale_tc(raw, dis2d)

    return _final_tc(raw, dis2d)
